# baseline (speedup 1.0000x reference)
import functools

import jax
import jax.numpy as jnp
from jax import lax
from jax.experimental import pallas as pl
from jax.experimental.pallas import tpu as pltpu
from jax.experimental.pallas import tpu_sc as plsc

N = 10000
E = 160000
D = 256
B = 16
OUT = 16
RB = 1000
NB = N // RB

NSC = 2
NTL = 16
NW = NSC * NTL
RPT = 312
RLAST = N - RPT * (NW - 1)
TRASH = 328
ACC_ROWS = 336
DB = 128
CAP = E + DB
SCH = 2000
NSCH = E // SCH
NGRP = SCH // 16
ERC = 80
_SC_MESH = dict(core_axis_name="c", subcore_axis_name="s",
                num_cores=NSC, num_subcores=NTL)


def _embed_body(ids_hbm, pe_hbm, emb_hbm, p16_hbm, out_hbm,
                idxv, rows, pev, p16v, sem):
    c = lax.axis_index("c")
    s = lax.axis_index("s")
    wid = s * NSC + c
    start = jnp.minimum(wid * (4 * ERC), N - 4 * ERC)
    pltpu.sync_copy(p16_hbm, p16v)
    p0 = p16v[...]
    for j in range(4):
        o = start + j * ERC
        pltpu.sync_copy(ids_hbm.at[pl.ds(o, ERC)], idxv)
        pltpu.async_copy(emb_hbm.at[idxv], rows, sem).wait()
        pltpu.sync_copy(pe_hbm.at[pl.ds(o, ERC)], pev)

        def addrow(r, _):
            for k in range(D // 16):
                sl = pl.ds(k * 16, 16)
                rows[r, sl] = rows[r, sl] + p0 * pev[r, sl]
            return 0
        lax.fori_loop(0, ERC, addrow, 0)
        pltpu.sync_copy(rows, out_hbm.at[pl.ds(o, ERC)])


@functools.partial(
    pl.kernel,
    out_type=jax.ShapeDtypeStruct((N, D), jnp.float32),
    mesh=plsc.VectorSubcoreMesh(**_SC_MESH),
    scratch_types=[
        pltpu.VMEM((ERC,), jnp.int32),
        pltpu.VMEM((ERC, D), jnp.float32),
        pltpu.VMEM((ERC, D), jnp.float32),
        pltpu.VMEM((16,), jnp.float32),
        pltpu.SemaphoreType.DMA,
    ],
)
def _embed(*args):
    _embed_body(*args)


_GDN = lax.GatherDimensionNumbers(
    offset_dims=(), collapsed_slice_dims=(0,), start_index_map=(0,))


def _dg(vec, idx):
    return lax.gather(vec, idx[:, None], _GDN, (1,),
                      mode=lax.GatherScatterMode.PROMISE_IN_BOUNDS)


def _prep_body(src_hbm, dst_hbm, w_hbm,
               ldst_hbm, srcl_hbm, wl_hbm, cnt_hbm,
               dstb, srcb, wch, ldsel, srcsel, wsel, cbuf):
    c = lax.axis_index("c")
    s = lax.axis_index("s")
    wid = c * NTL + s
    lo = wid * RPT
    hi = lo + jnp.where(wid == NW - 1, RLAST, RPT)
    lbase = wid * CAP
    lane = lax.iota(jnp.int32, 16)

    def flush(carry):
        ptr, done = carry
        o = lbase + done * DB
        pltpu.sync_copy(ldsel.at[pl.ds(0, DB)], ldst_hbm.at[pl.ds(o, DB)])
        pltpu.sync_copy(srcsel.at[pl.ds(0, DB)], srcl_hbm.at[pl.ds(o, DB)])
        pltpu.sync_copy(wsel.at[pl.ds(0, DB)], wl_hbm.at[pl.ds(o, DB)])
        ldsel[pl.ds(0, 16)] = ldsel[pl.ds(DB, 16)]
        srcsel[pl.ds(0, 16)] = srcsel[pl.ds(DB, 16)]
        wsel[pl.ds(0, 16)] = wsel[pl.ds(DB, 16)]
        return ptr - DB, done + 1

    def scan_chunk(ch, carry):
        pltpu.sync_copy(dst_hbm.at[pl.ds(ch * SCH, SCH)], dstb)
        pltpu.sync_copy(src_hbm.at[pl.ds(ch * SCH, SCH)], srcb)
        pltpu.sync_copy(w_hbm.at[pl.ds(ch * SCH, SCH)], wch)

        def grp(g, carry):
            sl16 = pl.ds(g * 16, 16)
            v = dstb[sl16]
            m = (v >= lo) & (v < hi)

            x = jnp.where(m, 1, 0)
            for k in (1, 2, 4, 8):
                sh = _dg(x, jnp.maximum(lane - k, 0))
                x = x + jnp.where(lane >= k, sh, 0)
            cnt = x[15]

            def sel(carry):
                ptr, done = carry
                tgt = lane + 1
                pos = jnp.zeros((16,), jnp.int32)
                for st in (8, 4, 2, 1):
                    cand = pos + st
                    xv = _dg(x, cand - 1)
                    pos = jnp.where(xv < tgt, cand, pos)
                srci = jnp.minimum(pos, 15)
                ldsel[pl.ds(ptr, 16)] = _dg(v, srci) - lo
                srcsel[pl.ds(ptr, 16)] = _dg(srcb[sl16], srci)
                wsel[pl.ds(ptr, 16)] = _dg(wch[sl16], srci)
                ptr = ptr + cnt
                return lax.cond(ptr >= DB, flush, lambda cc: cc, (ptr, done))
            return lax.cond(cnt > 0, sel, lambda cc: cc, carry)
        return lax.fori_loop(0, NGRP, grp, carry)

    ptr, done = lax.fori_loop(0, NSCH, scan_chunk, (0, 0))

    trash_l = jnp.full((16,), TRASH, jnp.int32)
    zero_i = jnp.zeros((16,), jnp.int32)
    zero_f = jnp.zeros((16,), jnp.float32)
    for t in range(DB // 16):
        ldsel[pl.ds(ptr + t * 16, 16)] = trash_l
        srcsel[pl.ds(ptr + t * 16, 16)] = zero_i
        wsel[pl.ds(ptr + t * 16, 16)] = zero_f
    _, done = flush((ptr, done))

    cbuf[...] = jnp.full((16,), done, jnp.int32)
    pltpu.sync_copy(cbuf, cnt_hbm.at[pl.ds(wid * 16, 16)])


@functools.partial(
    pl.kernel,
    out_type=(
        jax.ShapeDtypeStruct((NW * CAP,), jnp.int32),
        jax.ShapeDtypeStruct((NW * CAP,), jnp.int32),
        jax.ShapeDtypeStruct((NW * CAP,), jnp.float32),
        jax.ShapeDtypeStruct((NW * 16,), jnp.int32),
    ),
    mesh=plsc.VectorSubcoreMesh(**_SC_MESH),
    scratch_types=[
        pltpu.VMEM((SCH,), jnp.int32),
        pltpu.VMEM((SCH,), jnp.int32),
        pltpu.VMEM((SCH,), jnp.float32),
        pltpu.VMEM((DB + DB + 16,), jnp.int32),
        pltpu.VMEM((DB + DB + 16,), jnp.int32),
        pltpu.VMEM((DB + DB + 16,), jnp.float32),
        pltpu.VMEM((16,), jnp.int32),
    ],
)
def _edge_prep(*args):
    _prep_body(*args)


def _apply_body(h_hbm, ldst_hbm, srcl_hbm, wl_hbm, cnt_hbm, out_hbm,
                lbuf, sbuf, wbuf, cbuf, rows, acc, sem):
    c = lax.axis_index("c")
    s = lax.axis_index("s")
    wid = c * NTL + s
    lo = wid * RPT
    lbase = wid * CAP

    pltpu.sync_copy(h_hbm.at[pl.ds(lo, RPT)], acc.at[pl.ds(0, RPT)])

    @pl.when(wid == NW - 1)
    def _():
        pltpu.sync_copy(h_hbm.at[pl.ds(lo + RPT, RLAST - RPT)],
                        acc.at[pl.ds(RPT, RLAST - RPT)])

    pltpu.sync_copy(cnt_hbm.at[pl.ds(wid * 16, 16)], cbuf)
    nb = cbuf[...][0]

    def block(b, _):
        o = lbase + b * DB
        pltpu.sync_copy(ldst_hbm.at[pl.ds(o, DB)], lbuf)
        pltpu.sync_copy(wl_hbm.at[pl.ds(o, DB)], wbuf)
        pltpu.sync_copy(srcl_hbm.at[pl.ds(o, DB)], sbuf)
        pltpu.async_copy(h_hbm.at[sbuf], rows, sem).wait()

        def acc16(gg, _):
            wg = wbuf[pl.ds(gg * 16, 16)]
            lg = lbuf[pl.ds(gg * 16, 16)]
            for j in range(16):
                r = lg[j]
                w = wg[j]
                e = gg * 16 + j
                for k in range(D // 16):
                    sl = pl.ds(k * 16, 16)
                    acc[r, sl] = acc[r, sl] + rows[e, sl] * w
            return 0
        lax.fori_loop(0, DB // 16, acc16, 0)
        return 0
    lax.fori_loop(0, nb, block, 0)

    pltpu.sync_copy(acc.at[pl.ds(0, RPT)], out_hbm.at[pl.ds(lo, RPT)])

    @pl.when(wid == NW - 1)
    def _():
        pltpu.sync_copy(acc.at[pl.ds(RPT, RLAST - RPT)],
                        out_hbm.at[pl.ds(lo + RPT, RLAST - RPT)])


@functools.partial(
    pl.kernel,
    out_type=jax.ShapeDtypeStruct((N, D), jnp.float32),
    mesh=plsc.VectorSubcoreMesh(**_SC_MESH),
    scratch_types=[
        pltpu.VMEM((DB,), jnp.int32),
        pltpu.VMEM((DB,), jnp.int32),
        pltpu.VMEM((DB,), jnp.float32),
        pltpu.VMEM((16,), jnp.int32),
        pltpu.VMEM((DB, D), jnp.float32),
        pltpu.VMEM((ACC_ROWS, D), jnp.float32),
        pltpu.SemaphoreType.DMA,
    ],
)
def _spmm_apply(*args):
    _apply_body(*args)


def _mlp_stats_body(agg_ref, w0_ref, b0_ref, w1_ref, b1_ref, x_ref, stats_ref):
    a = agg_ref[...]
    t = jnp.maximum(jnp.dot(a, w0_ref[...], preferred_element_type=jnp.float32)
                    + b0_ref[...], 0.0)
    y = jnp.dot(t, w1_ref[...], preferred_element_type=jnp.float32) + b1_ref[...]
    y = jnp.where(y > 0, y, 0.01 * y)
    x_ref[...] = y

    @pl.when(pl.program_id(0) == 0)
    def _():
        stats_ref[...] = jnp.zeros_like(stats_ref)
    stats_ref[0:1, :] = stats_ref[0:1, :] + jnp.sum(y, axis=0, keepdims=True)
    stats_ref[1:2, :] = stats_ref[1:2, :] + jnp.sum(y * y, axis=0, keepdims=True)


def _mlp_stats(agg, w0, b0, w1, b1):
    return pl.pallas_call(
        _mlp_stats_body,
        grid=(NB,),
        in_specs=[
            pl.BlockSpec((RB, D), lambda i: (i, 0)),
            pl.BlockSpec((D, D), lambda i: (0, 0)),
            pl.BlockSpec((1, D), lambda i: (0, 0)),
            pl.BlockSpec((D, D), lambda i: (0, 0)),
            pl.BlockSpec((1, D), lambda i: (0, 0)),
        ],
        out_specs=[
            pl.BlockSpec((RB, D), lambda i: (i, 0)),
            pl.BlockSpec((8, D), lambda i: (0, 0)),
        ],
        out_shape=[
            jax.ShapeDtypeStruct((N, D), jnp.float32),
            jax.ShapeDtypeStruct((8, D), jnp.float32),
        ],
    )(agg, w0, b0, w1, b1)


def _bn_elin_body(x_ref, stats_ref, gamma_ref, beta_ref, attw_ref, sc_ref,
                  gp1_ref, gp2_ref, h_ref, e_ref, emax_ref):
    mean = stats_ref[0:1, :] * (1.0 / N)
    var = stats_ref[1:2, :] * (1.0 / N) - mean * mean
    inv = lax.rsqrt(var + 1e-5)
    h = gamma_ref[...] * (x_ref[...] - mean) * inv + beta_ref[...]
    h_ref[...] = h
    e = jnp.dot(h, attw_ref[...], preferred_element_type=jnp.float32)
    e = (e + gp1_ref[...] * sc_ref[0:1, 0:1] + gp2_ref[...] * sc_ref[0:1, 1:2]
         + sc_ref[0:1, 2:3])
    e_ref[...] = e

    @pl.when(pl.program_id(0) == 0)
    def _():
        emax_ref[...] = jnp.full_like(emax_ref, -jnp.inf)
    emax_ref[...] = jnp.maximum(emax_ref[...], jnp.max(e))


def _bn_elin(x, stats, gamma, beta, attw, sc, gp1, gp2):
    return pl.pallas_call(
        _bn_elin_body,
        grid=(NB,),
        in_specs=[
            pl.BlockSpec((RB, D), lambda i: (i, 0)),
            pl.BlockSpec((8, D), lambda i: (0, 0)),
            pl.BlockSpec((1, D), lambda i: (0, 0)),
            pl.BlockSpec((1, D), lambda i: (0, 0)),
            pl.BlockSpec((D, 1), lambda i: (0, 0)),
            pl.BlockSpec((1, 128), lambda i: (0, 0)),
            pl.BlockSpec((RB, 1), lambda i: (i, 0)),
            pl.BlockSpec((RB, 1), lambda i: (i, 0)),
        ],
        out_specs=[
            pl.BlockSpec((RB, D), lambda i: (i, 0)),
            pl.BlockSpec((RB, 1), lambda i: (i, 0)),
            pl.BlockSpec((1, 1), lambda i: (0, 0)),
        ],
        out_shape=[
            jax.ShapeDtypeStruct((N, D), jnp.float32),
            jax.ShapeDtypeStruct((N, 1), jnp.float32),
            jax.ShapeDtypeStruct((1, 1), jnp.float32),
        ],
    )(x, stats, gamma, beta, attw, sc, gp1, gp2)


def _elin_body(h_ref, attw_ref, sc_ref, gp1_ref, gp2_ref, e_ref, emax_ref):
    e = jnp.dot(h_ref[...], attw_ref[...], preferred_element_type=jnp.float32)
    e = (e + gp1_ref[...] * sc_ref[0:1, 0:1] + gp2_ref[...] * sc_ref[0:1, 1:2]
         + sc_ref[0:1, 2:3])
    e_ref[...] = e

    @pl.when(pl.program_id(0) == 0)
    def _():
        emax_ref[...] = jnp.full_like(emax_ref, -jnp.inf)
    emax_ref[...] = jnp.maximum(emax_ref[...], jnp.max(e))


def _elin(h, attw, sc, gp1, gp2):
    return pl.pallas_call(
        _elin_body,
        grid=(NB,),
        in_specs=[
            pl.BlockSpec((RB, D), lambda i: (i, 0)),
            pl.BlockSpec((D, 1), lambda i: (0, 0)),
            pl.BlockSpec((1, 128), lambda i: (0, 0)),
            pl.BlockSpec((RB, 1), lambda i: (i, 0)),
            pl.BlockSpec((RB, 1), lambda i: (i, 0)),
        ],
        out_specs=[
            pl.BlockSpec((RB, 1), lambda i: (i, 0)),
            pl.BlockSpec((1, 1), lambda i: (0, 0)),
        ],
        out_shape=[
            jax.ShapeDtypeStruct((N, 1), jnp.float32),
            jax.ShapeDtypeStruct((1, 1), jnp.float32),
        ],
    )(h, attw, sc, gp1, gp2)


def _pool_body(gid_ref, h0_ref, h1_ref, h2_ref, e0_ref, e1_ref, e2_ref,
               m0_ref, m1_ref, m2_ref,
               p0_ref, p1_ref, p2_ref, r0_ref, r1_ref, r2_ref):
    gid = gid_ref[0]
    oh = (gid == lax.broadcasted_iota(jnp.int32, (B, RB), 0)).astype(jnp.float32)

    @pl.when(pl.program_id(0) == 0)
    def _():
        for ref in (p0_ref, p1_ref, p2_ref, r0_ref, r1_ref, r2_ref):
            ref[...] = jnp.zeros_like(ref)

    for h_ref, e_ref, m_ref, p_ref, r_ref in (
            (h0_ref, e0_ref, m0_ref, p0_ref, r0_ref),
            (h1_ref, e1_ref, m1_ref, p1_ref, r1_ref),
            (h2_ref, e2_ref, m2_ref, p2_ref, r2_ref)):
        ee = jnp.exp(e_ref[...] - m_ref[...])
        eh = ee * h_ref[...]
        p_ref[...] = p_ref[...] + jnp.dot(oh, eh, preferred_element_type=jnp.float32)
        eb = jnp.broadcast_to(ee, (RB, 128))
        r_ref[...] = r_ref[...] + jnp.dot(oh, eb, preferred_element_type=jnp.float32)


def _pool(gid3, hs, es, ms):
    blk = lambda shape: pl.BlockSpec(shape, lambda i: (i, 0))
    cst = lambda shape: pl.BlockSpec(shape, lambda i: (0, 0))
    return pl.pallas_call(
        _pool_body,
        grid=(NB,),
        in_specs=[
            pl.BlockSpec((1, 1, RB), lambda i: (i, 0, 0)),
            blk((RB, D)), blk((RB, D)), blk((RB, D)),
            blk((RB, 1)), blk((RB, 1)), blk((RB, 1)),
            cst((1, 1)), cst((1, 1)), cst((1, 1)),
        ],
        out_specs=[cst((B, D)), cst((B, D)), cst((B, D)),
                   cst((B, 128)), cst((B, 128)), cst((B, 128))],
        out_shape=[jax.ShapeDtypeStruct((B, D), jnp.float32)] * 3
                  + [jax.ShapeDtypeStruct((B, 128), jnp.float32)] * 3,
    )(gid3, *hs, *es, *ms)


def _head_body(p0_ref, p1_ref, p2_ref, r0_ref, r1_ref, r2_ref,
               w0_ref, w1_ref, w2_ref, pb_ref,
               score_ref, o0_ref, o1_ref, o2_ref):
    score = jnp.zeros((B, OUT), jnp.float32)
    for i, (p_ref, r_ref, w_ref, o_ref) in enumerate(
            ((p0_ref, r0_ref, w0_ref, o0_ref),
             (p1_ref, r1_ref, w1_ref, o1_ref),
             (p2_ref, r2_ref, w2_ref, o2_ref))):
        pooled = p_ref[...] / (r_ref[:, 0:1] + 1e-10)
        o_ref[...] = pooled
        score = score + jnp.dot(pooled, w_ref[...],
                                preferred_element_type=jnp.float32) \
            + pb_ref[i:i + 1, :]
    score_ref[...] = score


def _head(praws, rsums, predws, predb):
    full = lambda shape: pl.BlockSpec(shape, lambda: (0, 0))
    return pl.pallas_call(
        _head_body,
        in_specs=[full((B, D))] * 3 + [full((B, 128))] * 3
                 + [full((D, OUT))] * 3 + [full((3, OUT))],
        out_specs=[full((B, OUT))] + [full((B, D))] * 3,
        out_shape=[jax.ShapeDtypeStruct((B, OUT), jnp.float32)]
                  + [jax.ShapeDtypeStruct((B, D), jnp.float32)] * 3,
    )(*praws, *rsums, *predws, predb)


def kernel(node_ids, pos_enc, edge_index, edge_weights, graph_ids, elem_gp1,
           elem_gp2, word_emb, pos, gnn_W0, gnn_b0, gnn_W1, gnn_b1, bn_gamma,
           bn_beta, att_W, att_b, pred_W, pred_b):
    src = edge_index[0]
    dst = edge_index[1]
    gp1 = elem_gp1.reshape(N, 1)
    gp2 = elem_gp2.reshape(N, 1)
    gid3 = graph_ids.reshape(NB, 1, RB)

    def att_params(l):
        attw = att_W[l, :D, :]
        sc = jnp.zeros((1, 128), jnp.float32)
        sc = sc.at[0, 0].set(att_W[l, D, 0])
        sc = sc.at[0, 1].set(att_W[l, D + 1, 0])
        sc = sc.at[0, 2].set(att_b[l, 0])
        return attw, sc

    pos16 = jnp.broadcast_to(pos[0:1], (16,))
    h = _embed(node_ids, pos_enc, word_emb, pos16)
    elist_ldst, elist_src, elist_w, elist_cnt = _edge_prep(src, dst,
                                                           edge_weights)

    attw0, sc0 = att_params(0)
    e0, m0 = _elin(h, attw0, sc0, gp1, gp2)

    hs, es, ms = [h], [e0], [m0]
    for l in range(2):
        agg = _spmm_apply(h, elist_ldst, elist_src, elist_w, elist_cnt)

        x, stats = _mlp_stats(agg, gnn_W0[l], gnn_b0[l].reshape(1, D),
                              gnn_W1[l], gnn_b1[l].reshape(1, D))
        attw, sc = att_params(l + 1)
        h, e, m = _bn_elin(x, stats, bn_gamma[l].reshape(1, D),
                           bn_beta[l].reshape(1, D), attw, sc, gp1, gp2)
        hs.append(h); es.append(e); ms.append(m)

    p0, p1, p2, r0, r1, r2 = _pool(gid3, hs, es, ms)
    score, o0, o1, o2 = _head((p0, p1, p2), (r0, r1, r2),
                              (pred_W[0], pred_W[1], pred_W[2]), pred_b)
    return (score, o0, o1, o2)


# speedup vs baseline: 1.0099x; 1.0099x over previous
import functools

import jax
import jax.numpy as jnp
from jax import lax
from jax.experimental import pallas as pl
from jax.experimental.pallas import tpu as pltpu
from jax.experimental.pallas import tpu_sc as plsc

N = 10000
E = 160000
D = 256
B = 16
OUT = 16
RB = 1000
NB = N // RB

NSC = 2
NTL = 16
NW = NSC * NTL
RPT = 312
RLAST = N - RPT * (NW - 1)
TRASH = 328
ACC_ROWS = 336
DB = 128
DBA = 64
CAP = E + DB
SCH = 2000
NSCH = E // SCH
NGRP = SCH // 16
ERC = 80
_SC_MESH = dict(core_axis_name="c", subcore_axis_name="s",
                num_cores=NSC, num_subcores=NTL)


def _embed_body(ids_hbm, pe_hbm, emb_hbm, p16_hbm, out_hbm,
                idxv, rows, pev, p16v, sem):
    c = lax.axis_index("c")
    s = lax.axis_index("s")
    wid = s * NSC + c
    start = jnp.minimum(wid * (4 * ERC), N - 4 * ERC)
    pltpu.sync_copy(p16_hbm, p16v)
    p0 = p16v[...]
    for j in range(4):
        o = start + j * ERC
        pltpu.sync_copy(ids_hbm.at[pl.ds(o, ERC)], idxv)
        pltpu.async_copy(emb_hbm.at[idxv], rows, sem).wait()
        pltpu.sync_copy(pe_hbm.at[pl.ds(o, ERC)], pev)

        def addrow(r, _):
            for k in range(D // 16):
                sl = pl.ds(k * 16, 16)
                rows[r, sl] = rows[r, sl] + p0 * pev[r, sl]
            return 0
        lax.fori_loop(0, ERC, addrow, 0)
        pltpu.sync_copy(rows, out_hbm.at[pl.ds(o, ERC)])


@functools.partial(
    pl.kernel,
    out_type=jax.ShapeDtypeStruct((N, D), jnp.float32),
    mesh=plsc.VectorSubcoreMesh(**_SC_MESH),
    scratch_types=[
        pltpu.VMEM((ERC,), jnp.int32),
        pltpu.VMEM((ERC, D), jnp.float32),
        pltpu.VMEM((ERC, D), jnp.float32),
        pltpu.VMEM((16,), jnp.float32),
        pltpu.SemaphoreType.DMA,
    ],
)
def _embed(*args):
    _embed_body(*args)


_GDN = lax.GatherDimensionNumbers(
    offset_dims=(), collapsed_slice_dims=(0,), start_index_map=(0,))


def _dg(vec, idx):
    return lax.gather(vec, idx[:, None], _GDN, (1,),
                      mode=lax.GatherScatterMode.PROMISE_IN_BOUNDS)


def _prep_body(src_hbm, dst_hbm, w_hbm,
               ldst_hbm, srcl_hbm, wl_hbm, cnt_hbm,
               dstb, srcb, wch, ldsel, srcsel, wsel, cbuf):
    c = lax.axis_index("c")
    s = lax.axis_index("s")
    wid = c * NTL + s
    lo = wid * RPT
    hi = lo + jnp.where(wid == NW - 1, RLAST, RPT)
    lbase = wid * CAP
    lane = lax.iota(jnp.int32, 16)

    def flush(carry):
        ptr, done = carry
        o = lbase + done * DB
        pltpu.sync_copy(ldsel.at[pl.ds(0, DB)], ldst_hbm.at[pl.ds(o, DB)])
        pltpu.sync_copy(srcsel.at[pl.ds(0, DB)], srcl_hbm.at[pl.ds(o, DB)])
        pltpu.sync_copy(wsel.at[pl.ds(0, DB)], wl_hbm.at[pl.ds(o, DB)])
        ldsel[pl.ds(0, 16)] = ldsel[pl.ds(DB, 16)]
        srcsel[pl.ds(0, 16)] = srcsel[pl.ds(DB, 16)]
        wsel[pl.ds(0, 16)] = wsel[pl.ds(DB, 16)]
        return ptr - DB, done + 1

    def scan_chunk(ch, carry):
        pltpu.sync_copy(dst_hbm.at[pl.ds(ch * SCH, SCH)], dstb)
        pltpu.sync_copy(src_hbm.at[pl.ds(ch * SCH, SCH)], srcb)
        pltpu.sync_copy(w_hbm.at[pl.ds(ch * SCH, SCH)], wch)

        def grp(g, carry):
            sl16 = pl.ds(g * 16, 16)
            v = dstb[sl16]
            m = (v >= lo) & (v < hi)

            x = jnp.where(m, 1, 0)
            for k in (1, 2, 4, 8):
                sh = _dg(x, jnp.maximum(lane - k, 0))
                x = x + jnp.where(lane >= k, sh, 0)
            cnt = x[15]

            def sel(carry):
                ptr, done = carry
                tgt = lane + 1
                pos = jnp.zeros((16,), jnp.int32)
                for st in (8, 4, 2, 1):
                    cand = pos + st
                    xv = _dg(x, cand - 1)
                    pos = jnp.where(xv < tgt, cand, pos)
                srci = jnp.minimum(pos, 15)
                ldsel[pl.ds(ptr, 16)] = _dg(v, srci) - lo
                srcsel[pl.ds(ptr, 16)] = _dg(srcb[sl16], srci)
                wsel[pl.ds(ptr, 16)] = _dg(wch[sl16], srci)
                ptr = ptr + cnt
                return lax.cond(ptr >= DB, flush, lambda cc: cc, (ptr, done))
            return lax.cond(cnt > 0, sel, lambda cc: cc, carry)
        return lax.fori_loop(0, NGRP, grp, carry)

    ptr, done = lax.fori_loop(0, NSCH, scan_chunk, (0, 0))

    trash_l = jnp.full((16,), TRASH, jnp.int32)
    zero_i = jnp.zeros((16,), jnp.int32)
    zero_f = jnp.zeros((16,), jnp.float32)
    for t in range(DB // 16):
        ldsel[pl.ds(ptr + t * 16, 16)] = trash_l
        srcsel[pl.ds(ptr + t * 16, 16)] = zero_i
        wsel[pl.ds(ptr + t * 16, 16)] = zero_f
    _, done = flush((ptr, done))

    cbuf[...] = jnp.full((16,), done, jnp.int32)
    pltpu.sync_copy(cbuf, cnt_hbm.at[pl.ds(wid * 16, 16)])


@functools.partial(
    pl.kernel,
    out_type=(
        jax.ShapeDtypeStruct((NW * CAP,), jnp.int32),
        jax.ShapeDtypeStruct((NW * CAP,), jnp.int32),
        jax.ShapeDtypeStruct((NW * CAP,), jnp.float32),
        jax.ShapeDtypeStruct((NW * 16,), jnp.int32),
    ),
    mesh=plsc.VectorSubcoreMesh(**_SC_MESH),
    scratch_types=[
        pltpu.VMEM((SCH,), jnp.int32),
        pltpu.VMEM((SCH,), jnp.int32),
        pltpu.VMEM((SCH,), jnp.float32),
        pltpu.VMEM((DB + DB + 16,), jnp.int32),
        pltpu.VMEM((DB + DB + 16,), jnp.int32),
        pltpu.VMEM((DB + DB + 16,), jnp.float32),
        pltpu.VMEM((16,), jnp.int32),
    ],
)
def _edge_prep(*args):
    _prep_body(*args)


def _apply_body(h_hbm, ldst_hbm, srcl_hbm, wl_hbm, cnt_hbm, out_hbm,
                lbuf0, sbuf0, wbuf0, rows0, lbuf1, sbuf1, wbuf1, rows1,
                cbuf, acc, seml0, semr0, seml1, semr1):
    c = lax.axis_index("c")
    s = lax.axis_index("s")
    wid = c * NTL + s
    lo = wid * RPT
    lbase = wid * CAP

    pltpu.sync_copy(h_hbm.at[pl.ds(lo, RPT)], acc.at[pl.ds(0, RPT)])

    @pl.when(wid == NW - 1)
    def _():
        pltpu.sync_copy(h_hbm.at[pl.ds(lo + RPT, RLAST - RPT)],
                        acc.at[pl.ds(RPT, RLAST - RPT)])

    pltpu.sync_copy(cnt_hbm.at[pl.ds(wid * 16, 16)], cbuf)
    nb2 = cbuf[...][0] * (DB // DBA)

    bufs = ((lbuf0, sbuf0, wbuf0, rows0, seml0, semr0),
            (lbuf1, sbuf1, wbuf1, rows1, seml1, semr1))

    def lists_refs(b, bs):
        o = lbase + b * DBA
        return ((ldst_hbm.at[pl.ds(o, DBA)], bs[0]),
                (srcl_hbm.at[pl.ds(o, DBA)], bs[1]),
                (wl_hbm.at[pl.ds(o, DBA)], bs[2]))

    def issue_lists(b, bs):
        for sref, dref in lists_refs(b, bs):
            pltpu.async_copy(sref, dref, bs[4])

    def wait_lists(b, bs):
        for sref, dref in lists_refs(b, bs):
            pltpu.make_async_copy(sref, dref, bs[4]).wait()

    def process(b, this, other):
        @pl.when(b + 1 < nb2)
        def _():
            wait_lists(b + 1, other)
            pltpu.async_copy(h_hbm.at[other[1]], other[3], other[5])

        pltpu.make_async_copy(h_hbm.at[this[1]], this[3], this[5]).wait()
        rows, lb, wb = this[3], this[0], this[2]

        def acc16(gg, _):
            wg = wb[pl.ds(gg * 16, 16)]
            lg = lb[pl.ds(gg * 16, 16)]
            for j in range(16):
                r = lg[j]
                w = wg[j]
                e = gg * 16 + j
                for k in range(D // 16):
                    sl = pl.ds(k * 16, 16)
                    acc[r, sl] = acc[r, sl] + rows[e, sl] * w
            return 0
        lax.fori_loop(0, DBA // 16, acc16, 0)

        @pl.when(b + 2 < nb2)
        def _():
            issue_lists(b + 2, this)

    for sref, dref in lists_refs(0, bufs[0]):
        pltpu.sync_copy(sref, dref)
    pltpu.async_copy(h_hbm.at[bufs[0][1]], bufs[0][3], bufs[0][5])

    @pl.when(nb2 > 1)
    def _():
        issue_lists(1, bufs[1])

    def pair(i, _):
        process(2 * i, bufs[0], bufs[1])

        @pl.when(2 * i + 1 < nb2)
        def _():
            process(2 * i + 1, bufs[1], bufs[0])
        return 0
    lax.fori_loop(0, (nb2 + 1) // 2, pair, 0)

    pltpu.sync_copy(acc.at[pl.ds(0, RPT)], out_hbm.at[pl.ds(lo, RPT)])

    @pl.when(wid == NW - 1)
    def _():
        pltpu.sync_copy(acc.at[pl.ds(RPT, RLAST - RPT)],
                        out_hbm.at[pl.ds(lo + RPT, RLAST - RPT)])


@functools.partial(
    pl.kernel,
    out_type=jax.ShapeDtypeStruct((N, D), jnp.float32),
    mesh=plsc.VectorSubcoreMesh(**_SC_MESH),
    scratch_types=[
        pltpu.VMEM((DBA,), jnp.int32),
        pltpu.VMEM((DBA,), jnp.int32),
        pltpu.VMEM((DBA,), jnp.float32),
        pltpu.VMEM((DBA, D), jnp.float32),
        pltpu.VMEM((DBA,), jnp.int32),
        pltpu.VMEM((DBA,), jnp.int32),
        pltpu.VMEM((DBA,), jnp.float32),
        pltpu.VMEM((DBA, D), jnp.float32),
        pltpu.VMEM((16,), jnp.int32),
        pltpu.VMEM((ACC_ROWS, D), jnp.float32),
        pltpu.SemaphoreType.DMA,
        pltpu.SemaphoreType.DMA,
        pltpu.SemaphoreType.DMA,
        pltpu.SemaphoreType.DMA,
    ],
)
def _spmm_apply(*args):
    _apply_body(*args)


def _mlp_stats_body(agg_ref, w0_ref, b0_ref, w1_ref, b1_ref, x_ref, stats_ref):
    a = agg_ref[...]
    t = jnp.maximum(jnp.dot(a, w0_ref[...], preferred_element_type=jnp.float32)
                    + b0_ref[...], 0.0)
    y = jnp.dot(t, w1_ref[...], preferred_element_type=jnp.float32) + b1_ref[...]
    y = jnp.where(y > 0, y, 0.01 * y)
    x_ref[...] = y

    @pl.when(pl.program_id(0) == 0)
    def _():
        stats_ref[...] = jnp.zeros_like(stats_ref)
    stats_ref[0:1, :] = stats_ref[0:1, :] + jnp.sum(y, axis=0, keepdims=True)
    stats_ref[1:2, :] = stats_ref[1:2, :] + jnp.sum(y * y, axis=0, keepdims=True)


def _mlp_stats(agg, w0, b0, w1, b1):
    return pl.pallas_call(
        _mlp_stats_body,
        grid=(NB,),
        in_specs=[
            pl.BlockSpec((RB, D), lambda i: (i, 0)),
            pl.BlockSpec((D, D), lambda i: (0, 0)),
            pl.BlockSpec((1, D), lambda i: (0, 0)),
            pl.BlockSpec((D, D), lambda i: (0, 0)),
            pl.BlockSpec((1, D), lambda i: (0, 0)),
        ],
        out_specs=[
            pl.BlockSpec((RB, D), lambda i: (i, 0)),
            pl.BlockSpec((8, D), lambda i: (0, 0)),
        ],
        out_shape=[
            jax.ShapeDtypeStruct((N, D), jnp.float32),
            jax.ShapeDtypeStruct((8, D), jnp.float32),
        ],
    )(agg, w0, b0, w1, b1)


def _bn_elin_body(x_ref, stats_ref, gamma_ref, beta_ref, attw_ref, sc_ref,
                  gp1_ref, gp2_ref, h_ref, e_ref, emax_ref):
    mean = stats_ref[0:1, :] * (1.0 / N)
    var = stats_ref[1:2, :] * (1.0 / N) - mean * mean
    inv = lax.rsqrt(var + 1e-5)
    h = gamma_ref[...] * (x_ref[...] - mean) * inv + beta_ref[...]
    h_ref[...] = h
    e = jnp.dot(h, attw_ref[...], preferred_element_type=jnp.float32)
    e = (e + gp1_ref[...] * sc_ref[0:1, 0:1] + gp2_ref[...] * sc_ref[0:1, 1:2]
         + sc_ref[0:1, 2:3])
    e_ref[...] = e

    @pl.when(pl.program_id(0) == 0)
    def _():
        emax_ref[...] = jnp.full_like(emax_ref, -jnp.inf)
    emax_ref[...] = jnp.maximum(emax_ref[...], jnp.max(e))


def _bn_elin(x, stats, gamma, beta, attw, sc, gp1, gp2):
    return pl.pallas_call(
        _bn_elin_body,
        grid=(NB,),
        in_specs=[
            pl.BlockSpec((RB, D), lambda i: (i, 0)),
            pl.BlockSpec((8, D), lambda i: (0, 0)),
            pl.BlockSpec((1, D), lambda i: (0, 0)),
            pl.BlockSpec((1, D), lambda i: (0, 0)),
            pl.BlockSpec((D, 1), lambda i: (0, 0)),
            pl.BlockSpec((1, 128), lambda i: (0, 0)),
            pl.BlockSpec((RB, 1), lambda i: (i, 0)),
            pl.BlockSpec((RB, 1), lambda i: (i, 0)),
        ],
        out_specs=[
            pl.BlockSpec((RB, D), lambda i: (i, 0)),
            pl.BlockSpec((RB, 1), lambda i: (i, 0)),
            pl.BlockSpec((1, 1), lambda i: (0, 0)),
        ],
        out_shape=[
            jax.ShapeDtypeStruct((N, D), jnp.float32),
            jax.ShapeDtypeStruct((N, 1), jnp.float32),
            jax.ShapeDtypeStruct((1, 1), jnp.float32),
        ],
    )(x, stats, gamma, beta, attw, sc, gp1, gp2)


def _elin_body(h_ref, attw_ref, sc_ref, gp1_ref, gp2_ref, e_ref, emax_ref):
    e = jnp.dot(h_ref[...], attw_ref[...], preferred_element_type=jnp.float32)
    e = (e + gp1_ref[...] * sc_ref[0:1, 0:1] + gp2_ref[...] * sc_ref[0:1, 1:2]
         + sc_ref[0:1, 2:3])
    e_ref[...] = e

    @pl.when(pl.program_id(0) == 0)
    def _():
        emax_ref[...] = jnp.full_like(emax_ref, -jnp.inf)
    emax_ref[...] = jnp.maximum(emax_ref[...], jnp.max(e))


def _elin(h, attw, sc, gp1, gp2):
    return pl.pallas_call(
        _elin_body,
        grid=(NB,),
        in_specs=[
            pl.BlockSpec((RB, D), lambda i: (i, 0)),
            pl.BlockSpec((D, 1), lambda i: (0, 0)),
            pl.BlockSpec((1, 128), lambda i: (0, 0)),
            pl.BlockSpec((RB, 1), lambda i: (i, 0)),
            pl.BlockSpec((RB, 1), lambda i: (i, 0)),
        ],
        out_specs=[
            pl.BlockSpec((RB, 1), lambda i: (i, 0)),
            pl.BlockSpec((1, 1), lambda i: (0, 0)),
        ],
        out_shape=[
            jax.ShapeDtypeStruct((N, 1), jnp.float32),
            jax.ShapeDtypeStruct((1, 1), jnp.float32),
        ],
    )(h, attw, sc, gp1, gp2)


def _pool_body(gid_ref, h0_ref, h1_ref, h2_ref, e0_ref, e1_ref, e2_ref,
               m0_ref, m1_ref, m2_ref,
               p0_ref, p1_ref, p2_ref, r0_ref, r1_ref, r2_ref):
    gid = gid_ref[0]
    oh = (gid == lax.broadcasted_iota(jnp.int32, (B, RB), 0)).astype(jnp.float32)

    @pl.when(pl.program_id(0) == 0)
    def _():
        for ref in (p0_ref, p1_ref, p2_ref, r0_ref, r1_ref, r2_ref):
            ref[...] = jnp.zeros_like(ref)

    for h_ref, e_ref, m_ref, p_ref, r_ref in (
            (h0_ref, e0_ref, m0_ref, p0_ref, r0_ref),
            (h1_ref, e1_ref, m1_ref, p1_ref, r1_ref),
            (h2_ref, e2_ref, m2_ref, p2_ref, r2_ref)):
        ee = jnp.exp(e_ref[...] - m_ref[...])
        eh = ee * h_ref[...]
        p_ref[...] = p_ref[...] + jnp.dot(oh, eh, preferred_element_type=jnp.float32)
        eb = jnp.broadcast_to(ee, (RB, 128))
        r_ref[...] = r_ref[...] + jnp.dot(oh, eb, preferred_element_type=jnp.float32)


def _pool(gid3, hs, es, ms):
    blk = lambda shape: pl.BlockSpec(shape, lambda i: (i, 0))
    cst = lambda shape: pl.BlockSpec(shape, lambda i: (0, 0))
    return pl.pallas_call(
        _pool_body,
        grid=(NB,),
        in_specs=[
            pl.BlockSpec((1, 1, RB), lambda i: (i, 0, 0)),
            blk((RB, D)), blk((RB, D)), blk((RB, D)),
            blk((RB, 1)), blk((RB, 1)), blk((RB, 1)),
            cst((1, 1)), cst((1, 1)), cst((1, 1)),
        ],
        out_specs=[cst((B, D)), cst((B, D)), cst((B, D)),
                   cst((B, 128)), cst((B, 128)), cst((B, 128))],
        out_shape=[jax.ShapeDtypeStruct((B, D), jnp.float32)] * 3
                  + [jax.ShapeDtypeStruct((B, 128), jnp.float32)] * 3,
    )(gid3, *hs, *es, *ms)


def _head_body(p0_ref, p1_ref, p2_ref, r0_ref, r1_ref, r2_ref,
               w0_ref, w1_ref, w2_ref, pb_ref,
               score_ref, o0_ref, o1_ref, o2_ref):
    score = jnp.zeros((B, OUT), jnp.float32)
    for i, (p_ref, r_ref, w_ref, o_ref) in enumerate(
            ((p0_ref, r0_ref, w0_ref, o0_ref),
             (p1_ref, r1_ref, w1_ref, o1_ref),
             (p2_ref, r2_ref, w2_ref, o2_ref))):
        pooled = p_ref[...] / (r_ref[:, 0:1] + 1e-10)
        o_ref[...] = pooled
        score = score + jnp.dot(pooled, w_ref[...],
                                preferred_element_type=jnp.float32) \
            + pb_ref[i:i + 1, :]
    score_ref[...] = score


def _head(praws, rsums, predws, predb):
    full = lambda shape: pl.BlockSpec(shape, lambda: (0, 0))
    return pl.pallas_call(
        _head_body,
        in_specs=[full((B, D))] * 3 + [full((B, 128))] * 3
                 + [full((D, OUT))] * 3 + [full((3, OUT))],
        out_specs=[full((B, OUT))] + [full((B, D))] * 3,
        out_shape=[jax.ShapeDtypeStruct((B, OUT), jnp.float32)]
                  + [jax.ShapeDtypeStruct((B, D), jnp.float32)] * 3,
    )(*praws, *rsums, *predws, predb)


def kernel(node_ids, pos_enc, edge_index, edge_weights, graph_ids, elem_gp1,
           elem_gp2, word_emb, pos, gnn_W0, gnn_b0, gnn_W1, gnn_b1, bn_gamma,
           bn_beta, att_W, att_b, pred_W, pred_b):
    src = edge_index[0]
    dst = edge_index[1]
    gp1 = elem_gp1.reshape(N, 1)
    gp2 = elem_gp2.reshape(N, 1)
    gid3 = graph_ids.reshape(NB, 1, RB)

    def att_params(l):
        attw = att_W[l, :D, :]
        sc = jnp.zeros((1, 128), jnp.float32)
        sc = sc.at[0, 0].set(att_W[l, D, 0])
        sc = sc.at[0, 1].set(att_W[l, D + 1, 0])
        sc = sc.at[0, 2].set(att_b[l, 0])
        return attw, sc

    pos16 = jnp.broadcast_to(pos[0:1], (16,))
    h = _embed(node_ids, pos_enc, word_emb, pos16)
    elist_ldst, elist_src, elist_w, elist_cnt = _edge_prep(src, dst,
                                                           edge_weights)

    attw0, sc0 = att_params(0)
    e0, m0 = _elin(h, attw0, sc0, gp1, gp2)

    hs, es, ms = [h], [e0], [m0]
    for l in range(2):
        agg = _spmm_apply(h, elist_ldst, elist_src, elist_w, elist_cnt)

        x, stats = _mlp_stats(agg, gnn_W0[l], gnn_b0[l].reshape(1, D),
                              gnn_W1[l], gnn_b1[l].reshape(1, D))
        attw, sc = att_params(l + 1)
        h, e, m = _bn_elin(x, stats, bn_gamma[l].reshape(1, D),
                           bn_beta[l].reshape(1, D), attw, sc, gp1, gp2)
        hs.append(h); es.append(e); ms.append(m)

    p0, p1, p2, r0, r1, r2 = _pool(gid3, hs, es, ms)
    score, o0, o1, o2 = _head((p0, p1, p2), (r0, r1, r2),
                              (pred_W[0], pred_W[1], pred_W[2]), pred_b)
    return (score, o0, o1, o2)


# speedup vs baseline: 1.6868x; 1.6703x over previous
import functools

import jax
import jax.numpy as jnp
from jax import lax
from jax.experimental import pallas as pl
from jax.experimental.pallas import tpu as pltpu
from jax.experimental.pallas import tpu_sc as plsc

N = 10000
E = 160000
D = 256
B = 16
OUT = 16
RB = 1000
NB = N // RB

NSC = 2
NTL = 16
NW = NSC * NTL
RPT = 312
RLAST = N - RPT * (NW - 1)
TRASH = 328
ACC_ROWS = 336
DB = 128
DBA = 64
CAP = E + DB
SCH = 2000
NSCH = E // SCH
NGRP = SCH // 16
ERC = 80
_SC_MESH = dict(core_axis_name="c", subcore_axis_name="s",
                num_cores=NSC, num_subcores=NTL)


def _embed_body(ids_hbm, pe_hbm, emb_hbm, p16_hbm, out_hbm,
                idxv, rows, pev, p16v, sem):
    c = lax.axis_index("c")
    s = lax.axis_index("s")
    wid = s * NSC + c
    start = jnp.minimum(wid * (4 * ERC), N - 4 * ERC)
    pltpu.sync_copy(p16_hbm, p16v)
    p0 = p16v[...]
    for j in range(4):
        o = start + j * ERC
        pltpu.sync_copy(ids_hbm.at[pl.ds(o, ERC)], idxv)
        pltpu.async_copy(emb_hbm.at[idxv], rows, sem).wait()
        pltpu.sync_copy(pe_hbm.at[pl.ds(o, ERC)], pev)

        def addrow(r, _):
            for k in range(D // 16):
                sl = pl.ds(k * 16, 16)
                rows[r, sl] = rows[r, sl] + p0 * pev[r, sl]
            return 0
        lax.fori_loop(0, ERC, addrow, 0)
        pltpu.sync_copy(rows, out_hbm.at[pl.ds(o, ERC)])


@functools.partial(
    pl.kernel,
    out_type=jax.ShapeDtypeStruct((N, D), jnp.float32),
    mesh=plsc.VectorSubcoreMesh(**_SC_MESH),
    scratch_types=[
        pltpu.VMEM((ERC,), jnp.int32),
        pltpu.VMEM((ERC, D), jnp.float32),
        pltpu.VMEM((ERC, D), jnp.float32),
        pltpu.VMEM((16,), jnp.float32),
        pltpu.SemaphoreType.DMA,
    ],
)
def _embed(*args):
    _embed_body(*args)


_GDN = lax.GatherDimensionNumbers(
    offset_dims=(), collapsed_slice_dims=(0,), start_index_map=(0,))


def _dg(vec, idx):
    return lax.gather(vec, idx[:, None], _GDN, (1,),
                      mode=lax.GatherScatterMode.PROMISE_IN_BOUNDS)


def _prep_body(src_hbm, dst_hbm, w_hbm,
               ldst_hbm, srcl_hbm, wl_hbm, cnt_hbm,
               dstb, srcb, wch, ldsel, srcsel, wsel, cbuf):
    c = lax.axis_index("c")
    s = lax.axis_index("s")
    wid = c * NTL + s
    lo = wid * RPT
    hi = lo + jnp.where(wid == NW - 1, RLAST, RPT)
    lbase = wid * CAP
    lane = lax.iota(jnp.int32, 16)

    def flush(carry):
        ptr, done = carry
        o = lbase + done * DB
        pltpu.sync_copy(ldsel.at[pl.ds(0, DB)], ldst_hbm.at[pl.ds(o, DB)])
        pltpu.sync_copy(srcsel.at[pl.ds(0, DB)], srcl_hbm.at[pl.ds(o, DB)])
        pltpu.sync_copy(wsel.at[pl.ds(0, DB)], wl_hbm.at[pl.ds(o, DB)])
        ldsel[pl.ds(0, 16)] = ldsel[pl.ds(DB, 16)]
        srcsel[pl.ds(0, 16)] = srcsel[pl.ds(DB, 16)]
        wsel[pl.ds(0, 16)] = wsel[pl.ds(DB, 16)]
        return ptr - DB, done + 1

    def scan_chunk(ch, carry):
        pltpu.sync_copy(dst_hbm.at[pl.ds(ch * SCH, SCH)], dstb)
        pltpu.sync_copy(src_hbm.at[pl.ds(ch * SCH, SCH)], srcb)
        pltpu.sync_copy(w_hbm.at[pl.ds(ch * SCH, SCH)], wch)

        def grp(g, carry):
            sl16 = pl.ds(g * 16, 16)
            v = dstb[sl16]
            m = (v >= lo) & (v < hi)

            x = jnp.where(m, 1, 0)
            for k in (1, 2, 4, 8):
                sh = _dg(x, jnp.maximum(lane - k, 0))
                x = x + jnp.where(lane >= k, sh, 0)
            cnt = x[15]

            def sel(carry):
                ptr, done = carry
                tgt = lane + 1
                pos = jnp.zeros((16,), jnp.int32)
                for st in (8, 4, 2, 1):
                    cand = pos + st
                    xv = _dg(x, cand - 1)
                    pos = jnp.where(xv < tgt, cand, pos)
                srci = jnp.minimum(pos, 15)
                ldsel[pl.ds(ptr, 16)] = _dg(v, srci) - lo
                srcsel[pl.ds(ptr, 16)] = _dg(srcb[sl16], srci)
                wsel[pl.ds(ptr, 16)] = _dg(wch[sl16], srci)
                ptr = ptr + cnt
                return lax.cond(ptr >= DB, flush, lambda cc: cc, (ptr, done))
            return lax.cond(cnt > 0, sel, lambda cc: cc, carry)
        return lax.fori_loop(0, NGRP, grp, carry)

    ptr, done = lax.fori_loop(0, NSCH, scan_chunk, (0, 0))

    trash_l = jnp.full((16,), TRASH, jnp.int32)
    zero_i = jnp.zeros((16,), jnp.int32)
    zero_f = jnp.zeros((16,), jnp.float32)
    for t in range(DB // 16):
        ldsel[pl.ds(ptr + t * 16, 16)] = trash_l
        srcsel[pl.ds(ptr + t * 16, 16)] = zero_i
        wsel[pl.ds(ptr + t * 16, 16)] = zero_f
    _, done = flush((ptr, done))

    cbuf[...] = jnp.full((16,), done, jnp.int32)
    pltpu.sync_copy(cbuf, cnt_hbm.at[pl.ds(wid * 16, 16)])


@functools.partial(
    pl.kernel,
    out_type=(
        jax.ShapeDtypeStruct((NW * CAP,), jnp.int32),
        jax.ShapeDtypeStruct((NW * CAP,), jnp.int32),
        jax.ShapeDtypeStruct((NW * CAP,), jnp.float32),
        jax.ShapeDtypeStruct((NW * 16,), jnp.int32),
    ),
    mesh=plsc.VectorSubcoreMesh(**_SC_MESH),
    scratch_types=[
        pltpu.VMEM((SCH,), jnp.int32),
        pltpu.VMEM((SCH,), jnp.int32),
        pltpu.VMEM((SCH,), jnp.float32),
        pltpu.VMEM((DB + DB + 16,), jnp.int32),
        pltpu.VMEM((DB + DB + 16,), jnp.int32),
        pltpu.VMEM((DB + DB + 16,), jnp.float32),
        pltpu.VMEM((16,), jnp.int32),
    ],
)
def _edge_prep(*args):
    _prep_body(*args)


def _apply_body(h_hbm, ldst_hbm, srcl_hbm, wl_hbm, cnt_hbm, out_hbm,
                lbuf0, sbuf0, wbuf0, rows0, lbuf1, sbuf1, wbuf1, rows1,
                cbuf, acc, seml0, semr0, seml1, semr1):
    c = lax.axis_index("c")
    s = lax.axis_index("s")
    wid = c * NTL + s
    lo = wid * RPT
    lbase = wid * CAP

    pltpu.sync_copy(h_hbm.at[pl.ds(lo, RPT)], acc.at[pl.ds(0, RPT)])

    @pl.when(wid == NW - 1)
    def _():
        pltpu.sync_copy(h_hbm.at[pl.ds(lo + RPT, RLAST - RPT)],
                        acc.at[pl.ds(RPT, RLAST - RPT)])

    pltpu.sync_copy(cnt_hbm.at[pl.ds(wid * 16, 16)], cbuf)
    nb2 = cbuf[...][0] * (DB // DBA)

    bufs = ((lbuf0, sbuf0, wbuf0, rows0, seml0, semr0),
            (lbuf1, sbuf1, wbuf1, rows1, seml1, semr1))

    def lists_refs(b, bs):
        o = lbase + b * DBA
        return ((ldst_hbm.at[pl.ds(o, DBA)], bs[0]),
                (srcl_hbm.at[pl.ds(o, DBA)], bs[1]),
                (wl_hbm.at[pl.ds(o, DBA)], bs[2]))

    def issue_lists(b, bs):
        for sref, dref in lists_refs(b, bs):
            pltpu.async_copy(sref, dref, bs[4])

    def wait_lists(b, bs):
        for sref, dref in lists_refs(b, bs):
            pltpu.make_async_copy(sref, dref, bs[4]).wait()

    def process(b, this, other):
        @pl.when(b + 1 < nb2)
        def _():
            wait_lists(b + 1, other)
            pltpu.async_copy(h_hbm.at[other[1]], other[3], other[5])

        pltpu.make_async_copy(h_hbm.at[this[1]], this[3], this[5]).wait()
        rows, lb, wb = this[3], this[0], this[2]

        def acc16(gg, _):
            wg = wb[pl.ds(gg * 16, 16)]
            lg = lb[pl.ds(gg * 16, 16)]
            for j in range(16):
                r = lg[j]
                w = wg[j]
                e = gg * 16 + j
                vals = [acc[r, pl.ds(k * 16, 16)] + rows[e, pl.ds(k * 16, 16)] * w
                        for k in range(D // 16)]
                for k in range(D // 16):
                    acc[r, pl.ds(k * 16, 16)] = vals[k]
            return 0
        lax.fori_loop(0, DBA // 16, acc16, 0)

        @pl.when(b + 2 < nb2)
        def _():
            issue_lists(b + 2, this)

    for sref, dref in lists_refs(0, bufs[0]):
        pltpu.sync_copy(sref, dref)
    pltpu.async_copy(h_hbm.at[bufs[0][1]], bufs[0][3], bufs[0][5])

    @pl.when(nb2 > 1)
    def _():
        issue_lists(1, bufs[1])

    def pair(i, _):
        process(2 * i, bufs[0], bufs[1])

        @pl.when(2 * i + 1 < nb2)
        def _():
            process(2 * i + 1, bufs[1], bufs[0])
        return 0
    lax.fori_loop(0, (nb2 + 1) // 2, pair, 0)

    pltpu.sync_copy(acc.at[pl.ds(0, RPT)], out_hbm.at[pl.ds(lo, RPT)])

    @pl.when(wid == NW - 1)
    def _():
        pltpu.sync_copy(acc.at[pl.ds(RPT, RLAST - RPT)],
                        out_hbm.at[pl.ds(lo + RPT, RLAST - RPT)])


@functools.partial(
    pl.kernel,
    out_type=jax.ShapeDtypeStruct((N, D), jnp.float32),
    mesh=plsc.VectorSubcoreMesh(**_SC_MESH),
    scratch_types=[
        pltpu.VMEM((DBA,), jnp.int32),
        pltpu.VMEM((DBA,), jnp.int32),
        pltpu.VMEM((DBA,), jnp.float32),
        pltpu.VMEM((DBA, D), jnp.float32),
        pltpu.VMEM((DBA,), jnp.int32),
        pltpu.VMEM((DBA,), jnp.int32),
        pltpu.VMEM((DBA,), jnp.float32),
        pltpu.VMEM((DBA, D), jnp.float32),
        pltpu.VMEM((16,), jnp.int32),
        pltpu.VMEM((ACC_ROWS, D), jnp.float32),
        pltpu.SemaphoreType.DMA,
        pltpu.SemaphoreType.DMA,
        pltpu.SemaphoreType.DMA,
        pltpu.SemaphoreType.DMA,
    ],
)
def _spmm_apply(*args):
    _apply_body(*args)


def _mlp_stats_body(agg_ref, w0_ref, b0_ref, w1_ref, b1_ref, x_ref, stats_ref):
    a = agg_ref[...]
    t = jnp.maximum(jnp.dot(a, w0_ref[...], preferred_element_type=jnp.float32)
                    + b0_ref[...], 0.0)
    y = jnp.dot(t, w1_ref[...], preferred_element_type=jnp.float32) + b1_ref[...]
    y = jnp.where(y > 0, y, 0.01 * y)
    x_ref[...] = y

    @pl.when(pl.program_id(0) == 0)
    def _():
        stats_ref[...] = jnp.zeros_like(stats_ref)
    stats_ref[0:1, :] = stats_ref[0:1, :] + jnp.sum(y, axis=0, keepdims=True)
    stats_ref[1:2, :] = stats_ref[1:2, :] + jnp.sum(y * y, axis=0, keepdims=True)


def _mlp_stats(agg, w0, b0, w1, b1):
    return pl.pallas_call(
        _mlp_stats_body,
        grid=(NB,),
        in_specs=[
            pl.BlockSpec((RB, D), lambda i: (i, 0)),
            pl.BlockSpec((D, D), lambda i: (0, 0)),
            pl.BlockSpec((1, D), lambda i: (0, 0)),
            pl.BlockSpec((D, D), lambda i: (0, 0)),
            pl.BlockSpec((1, D), lambda i: (0, 0)),
        ],
        out_specs=[
            pl.BlockSpec((RB, D), lambda i: (i, 0)),
            pl.BlockSpec((8, D), lambda i: (0, 0)),
        ],
        out_shape=[
            jax.ShapeDtypeStruct((N, D), jnp.float32),
            jax.ShapeDtypeStruct((8, D), jnp.float32),
        ],
    )(agg, w0, b0, w1, b1)


def _bn_elin_body(x_ref, stats_ref, gamma_ref, beta_ref, attw_ref, sc_ref,
                  gp1_ref, gp2_ref, h_ref, e_ref, emax_ref):
    mean = stats_ref[0:1, :] * (1.0 / N)
    var = stats_ref[1:2, :] * (1.0 / N) - mean * mean
    inv = lax.rsqrt(var + 1e-5)
    h = gamma_ref[...] * (x_ref[...] - mean) * inv + beta_ref[...]
    h_ref[...] = h
    e = jnp.dot(h, attw_ref[...], preferred_element_type=jnp.float32)
    e = (e + gp1_ref[...] * sc_ref[0:1, 0:1] + gp2_ref[...] * sc_ref[0:1, 1:2]
         + sc_ref[0:1, 2:3])
    e_ref[...] = e

    @pl.when(pl.program_id(0) == 0)
    def _():
        emax_ref[...] = jnp.full_like(emax_ref, -jnp.inf)
    emax_ref[...] = jnp.maximum(emax_ref[...], jnp.max(e))


def _bn_elin(x, stats, gamma, beta, attw, sc, gp1, gp2):
    return pl.pallas_call(
        _bn_elin_body,
        grid=(NB,),
        in_specs=[
            pl.BlockSpec((RB, D), lambda i: (i, 0)),
            pl.BlockSpec((8, D), lambda i: (0, 0)),
            pl.BlockSpec((1, D), lambda i: (0, 0)),
            pl.BlockSpec((1, D), lambda i: (0, 0)),
            pl.BlockSpec((D, 1), lambda i: (0, 0)),
            pl.BlockSpec((1, 128), lambda i: (0, 0)),
            pl.BlockSpec((RB, 1), lambda i: (i, 0)),
            pl.BlockSpec((RB, 1), lambda i: (i, 0)),
        ],
        out_specs=[
            pl.BlockSpec((RB, D), lambda i: (i, 0)),
            pl.BlockSpec((RB, 1), lambda i: (i, 0)),
            pl.BlockSpec((1, 1), lambda i: (0, 0)),
        ],
        out_shape=[
            jax.ShapeDtypeStruct((N, D), jnp.float32),
            jax.ShapeDtypeStruct((N, 1), jnp.float32),
            jax.ShapeDtypeStruct((1, 1), jnp.float32),
        ],
    )(x, stats, gamma, beta, attw, sc, gp1, gp2)


def _elin_body(h_ref, attw_ref, sc_ref, gp1_ref, gp2_ref, e_ref, emax_ref):
    e = jnp.dot(h_ref[...], attw_ref[...], preferred_element_type=jnp.float32)
    e = (e + gp1_ref[...] * sc_ref[0:1, 0:1] + gp2_ref[...] * sc_ref[0:1, 1:2]
         + sc_ref[0:1, 2:3])
    e_ref[...] = e

    @pl.when(pl.program_id(0) == 0)
    def _():
        emax_ref[...] = jnp.full_like(emax_ref, -jnp.inf)
    emax_ref[...] = jnp.maximum(emax_ref[...], jnp.max(e))


def _elin(h, attw, sc, gp1, gp2):
    return pl.pallas_call(
        _elin_body,
        grid=(NB,),
        in_specs=[
            pl.BlockSpec((RB, D), lambda i: (i, 0)),
            pl.BlockSpec((D, 1), lambda i: (0, 0)),
            pl.BlockSpec((1, 128), lambda i: (0, 0)),
            pl.BlockSpec((RB, 1), lambda i: (i, 0)),
            pl.BlockSpec((RB, 1), lambda i: (i, 0)),
        ],
        out_specs=[
            pl.BlockSpec((RB, 1), lambda i: (i, 0)),
            pl.BlockSpec((1, 1), lambda i: (0, 0)),
        ],
        out_shape=[
            jax.ShapeDtypeStruct((N, 1), jnp.float32),
            jax.ShapeDtypeStruct((1, 1), jnp.float32),
        ],
    )(h, attw, sc, gp1, gp2)


def _pool_body(gid_ref, h0_ref, h1_ref, h2_ref, e0_ref, e1_ref, e2_ref,
               m0_ref, m1_ref, m2_ref,
               p0_ref, p1_ref, p2_ref, r0_ref, r1_ref, r2_ref):
    gid = gid_ref[0]
    oh = (gid == lax.broadcasted_iota(jnp.int32, (B, RB), 0)).astype(jnp.float32)

    @pl.when(pl.program_id(0) == 0)
    def _():
        for ref in (p0_ref, p1_ref, p2_ref, r0_ref, r1_ref, r2_ref):
            ref[...] = jnp.zeros_like(ref)

    for h_ref, e_ref, m_ref, p_ref, r_ref in (
            (h0_ref, e0_ref, m0_ref, p0_ref, r0_ref),
            (h1_ref, e1_ref, m1_ref, p1_ref, r1_ref),
            (h2_ref, e2_ref, m2_ref, p2_ref, r2_ref)):
        ee = jnp.exp(e_ref[...] - m_ref[...])
        eh = ee * h_ref[...]
        p_ref[...] = p_ref[...] + jnp.dot(oh, eh, preferred_element_type=jnp.float32)
        eb = jnp.broadcast_to(ee, (RB, 128))
        r_ref[...] = r_ref[...] + jnp.dot(oh, eb, preferred_element_type=jnp.float32)


def _pool(gid3, hs, es, ms):
    blk = lambda shape: pl.BlockSpec(shape, lambda i: (i, 0))
    cst = lambda shape: pl.BlockSpec(shape, lambda i: (0, 0))
    return pl.pallas_call(
        _pool_body,
        grid=(NB,),
        in_specs=[
            pl.BlockSpec((1, 1, RB), lambda i: (i, 0, 0)),
            blk((RB, D)), blk((RB, D)), blk((RB, D)),
            blk((RB, 1)), blk((RB, 1)), blk((RB, 1)),
            cst((1, 1)), cst((1, 1)), cst((1, 1)),
        ],
        out_specs=[cst((B, D)), cst((B, D)), cst((B, D)),
                   cst((B, 128)), cst((B, 128)), cst((B, 128))],
        out_shape=[jax.ShapeDtypeStruct((B, D), jnp.float32)] * 3
                  + [jax.ShapeDtypeStruct((B, 128), jnp.float32)] * 3,
    )(gid3, *hs, *es, *ms)


def _head_body(p0_ref, p1_ref, p2_ref, r0_ref, r1_ref, r2_ref,
               w0_ref, w1_ref, w2_ref, pb_ref,
               score_ref, o0_ref, o1_ref, o2_ref):
    score = jnp.zeros((B, OUT), jnp.float32)
    for i, (p_ref, r_ref, w_ref, o_ref) in enumerate(
            ((p0_ref, r0_ref, w0_ref, o0_ref),
             (p1_ref, r1_ref, w1_ref, o1_ref),
             (p2_ref, r2_ref, w2_ref, o2_ref))):
        pooled = p_ref[...] / (r_ref[:, 0:1] + 1e-10)
        o_ref[...] = pooled
        score = score + jnp.dot(pooled, w_ref[...],
                                preferred_element_type=jnp.float32) \
            + pb_ref[i:i + 1, :]
    score_ref[...] = score


def _head(praws, rsums, predws, predb):
    full = lambda shape: pl.BlockSpec(shape, lambda: (0, 0))
    return pl.pallas_call(
        _head_body,
        in_specs=[full((B, D))] * 3 + [full((B, 128))] * 3
                 + [full((D, OUT))] * 3 + [full((3, OUT))],
        out_specs=[full((B, OUT))] + [full((B, D))] * 3,
        out_shape=[jax.ShapeDtypeStruct((B, OUT), jnp.float32)]
                  + [jax.ShapeDtypeStruct((B, D), jnp.float32)] * 3,
    )(*praws, *rsums, *predws, predb)


def kernel(node_ids, pos_enc, edge_index, edge_weights, graph_ids, elem_gp1,
           elem_gp2, word_emb, pos, gnn_W0, gnn_b0, gnn_W1, gnn_b1, bn_gamma,
           bn_beta, att_W, att_b, pred_W, pred_b):
    src = edge_index[0]
    dst = edge_index[1]
    gp1 = elem_gp1.reshape(N, 1)
    gp2 = elem_gp2.reshape(N, 1)
    gid3 = graph_ids.reshape(NB, 1, RB)

    def att_params(l):
        attw = att_W[l, :D, :]
        sc = jnp.zeros((1, 128), jnp.float32)
        sc = sc.at[0, 0].set(att_W[l, D, 0])
        sc = sc.at[0, 1].set(att_W[l, D + 1, 0])
        sc = sc.at[0, 2].set(att_b[l, 0])
        return attw, sc

    pos16 = jnp.broadcast_to(pos[0:1], (16,))
    h = _embed(node_ids, pos_enc, word_emb, pos16)
    elist_ldst, elist_src, elist_w, elist_cnt = _edge_prep(src, dst,
                                                           edge_weights)

    attw0, sc0 = att_params(0)
    e0, m0 = _elin(h, attw0, sc0, gp1, gp2)

    hs, es, ms = [h], [e0], [m0]
    for l in range(2):
        agg = _spmm_apply(h, elist_ldst, elist_src, elist_w, elist_cnt)

        x, stats = _mlp_stats(agg, gnn_W0[l], gnn_b0[l].reshape(1, D),
                              gnn_W1[l], gnn_b1[l].reshape(1, D))
        attw, sc = att_params(l + 1)
        h, e, m = _bn_elin(x, stats, bn_gamma[l].reshape(1, D),
                           bn_beta[l].reshape(1, D), attw, sc, gp1, gp2)
        hs.append(h); es.append(e); ms.append(m)

    p0, p1, p2, r0, r1, r2 = _pool(gid3, hs, es, ms)
    score, o0, o1, o2 = _head((p0, p1, p2), (r0, r1, r2),
                              (pred_W[0], pred_W[1], pred_W[2]), pred_b)
    return (score, o0, o1, o2)


# speedup vs baseline: 1.9135x; 1.1344x over previous
import functools

import jax
import jax.numpy as jnp
from jax import lax
from jax.experimental import pallas as pl
from jax.experimental.pallas import tpu as pltpu
from jax.experimental.pallas import tpu_sc as plsc

N = 10000
E = 160000
D = 256
B = 16
OUT = 16
RB = 1000
NB = N // RB

NSC = 2
NTL = 16
NW = NSC * NTL
RPT = 312
RLAST = N - RPT * (NW - 1)
TRASH = 328
ACC_ROWS = 336
DB = 128
DBA = 64
CAP = E + DB
SCH = 2000
NSCH = E // SCH
NGRP = SCH // 16
ERC = 80
_SC_MESH = dict(core_axis_name="c", subcore_axis_name="s",
                num_cores=NSC, num_subcores=NTL)


def _embed_body(ids_hbm, pe_hbm, emb_hbm, p16_hbm, out_hbm,
                idxv, rows, pev, p16v, sem):
    c = lax.axis_index("c")
    s = lax.axis_index("s")
    wid = s * NSC + c
    start = jnp.minimum(wid * (4 * ERC), N - 4 * ERC)
    pltpu.sync_copy(p16_hbm, p16v)
    p0 = p16v[...]
    for j in range(4):
        o = start + j * ERC
        pltpu.sync_copy(ids_hbm.at[pl.ds(o, ERC)], idxv)
        pltpu.async_copy(emb_hbm.at[idxv], rows, sem).wait()
        pltpu.sync_copy(pe_hbm.at[pl.ds(o, ERC)], pev)

        def addrow(r, _):
            for k in range(D // 16):
                sl = pl.ds(k * 16, 16)
                rows[r, sl] = rows[r, sl] + p0 * pev[r, sl]
            return 0
        lax.fori_loop(0, ERC, addrow, 0)
        pltpu.sync_copy(rows, out_hbm.at[pl.ds(o, ERC)])


@functools.partial(
    pl.kernel,
    out_type=jax.ShapeDtypeStruct((N, D), jnp.float32),
    mesh=plsc.VectorSubcoreMesh(**_SC_MESH),
    scratch_types=[
        pltpu.VMEM((ERC,), jnp.int32),
        pltpu.VMEM((ERC, D), jnp.float32),
        pltpu.VMEM((ERC, D), jnp.float32),
        pltpu.VMEM((16,), jnp.float32),
        pltpu.SemaphoreType.DMA,
    ],
)
def _embed(*args):
    _embed_body(*args)


_GDN = lax.GatherDimensionNumbers(
    offset_dims=(), collapsed_slice_dims=(0,), start_index_map=(0,))


def _dg(vec, idx):
    return lax.gather(vec, idx[:, None], _GDN, (1,),
                      mode=lax.GatherScatterMode.PROMISE_IN_BOUNDS)


def _prep_body(src_hbm, dst_hbm, w_hbm,
               ldst_hbm, srcl_hbm, wl_hbm, cnt_hbm,
               dstb, srcb, wch, dstb1, srcb1, wch1, ldsel, srcsel, wsel, cbuf,
               semc0, semc1):
    c = lax.axis_index("c")
    s = lax.axis_index("s")
    wid = c * NTL + s
    lo = wid * RPT
    hi = lo + jnp.where(wid == NW - 1, RLAST, RPT)
    lbase = wid * CAP
    lane = lax.iota(jnp.int32, 16)

    def flush(carry):
        ptr, done = carry
        o = lbase + done * DB
        pltpu.sync_copy(ldsel.at[pl.ds(0, DB)], ldst_hbm.at[pl.ds(o, DB)])
        pltpu.sync_copy(srcsel.at[pl.ds(0, DB)], srcl_hbm.at[pl.ds(o, DB)])
        pltpu.sync_copy(wsel.at[pl.ds(0, DB)], wl_hbm.at[pl.ds(o, DB)])
        ldsel[pl.ds(0, 16)] = ldsel[pl.ds(DB, 16)]
        srcsel[pl.ds(0, 16)] = srcsel[pl.ds(DB, 16)]
        wsel[pl.ds(0, 16)] = wsel[pl.ds(DB, 16)]
        return ptr - DB, done + 1

    shidx = [jnp.maximum(lane - k, 0) for k in (1, 2, 4, 8)]
    shmask = [lane >= k for k in (1, 2, 4, 8)]
    zero16 = jnp.zeros((16,), jnp.int32)

    def scan_chunk(ch, carry, db, sb, wb):
        def grp(g, carry):
            sl16 = pl.ds(g * 16, 16)
            v = db[sl16]
            m = (v >= lo) & (v < hi)

            x = jnp.where(m, 1, 0)
            for k in range(4):
                x = x + jnp.where(shmask[k], _dg(x, shidx[k]), 0)
            cnt = x[15]

            def sel(carry):
                ptr, done = carry
                tgt = lane + 1
                pos = zero16
                for st in (8, 4, 2, 1):
                    cand = pos + st
                    xv = _dg(x, cand - 1)
                    pos = jnp.where(xv < tgt, cand, pos)
                srci = jnp.minimum(pos, 15)
                ldsel[pl.ds(ptr, 16)] = _dg(v, srci) - lo
                srcsel[pl.ds(ptr, 16)] = _dg(sb[sl16], srci)
                wsel[pl.ds(ptr, 16)] = _dg(wb[sl16], srci)
                ptr = ptr + cnt
                return lax.cond(ptr >= DB, flush, lambda cc: cc, (ptr, done))
            return lax.cond(cnt > 0, sel, lambda cc: cc, carry)
        return lax.fori_loop(0, NGRP, grp, carry)

    cbufs = ((dstb, srcb, wch, semc0), (dstb1, srcb1, wch1, semc1))

    def chunk_refs(ch, cb):
        csl = pl.ds(ch * SCH, SCH)
        return ((dst_hbm.at[csl], cb[0]), (src_hbm.at[csl], cb[1]),
                (w_hbm.at[csl], cb[2]))

    def issue_chunk(ch, cb):
        for sref, dref in chunk_refs(ch, cb):
            pltpu.async_copy(sref, dref, cb[3])

    def wait_chunk(ch, cb):
        for sref, dref in chunk_refs(ch, cb):
            pltpu.make_async_copy(sref, dref, cb[3]).wait()

    for sref, dref in chunk_refs(0, cbufs[0]):
        pltpu.sync_copy(sref, dref)
    issue_chunk(1, cbufs[1])

    def pair(i, carry):
        carry = scan_chunk(2 * i, carry, cbufs[0][0], cbufs[0][1], cbufs[0][2])

        @pl.when(i < NSCH // 2 - 1)
        def _():
            issue_chunk(2 * i + 2, cbufs[0])
        wait_chunk(2 * i + 1, cbufs[1])
        carry = scan_chunk(2 * i + 1, carry,
                           cbufs[1][0], cbufs[1][1], cbufs[1][2])

        @pl.when(i < NSCH // 2 - 1)
        def _():
            issue_chunk(2 * i + 3, cbufs[1])
            wait_chunk(2 * i + 2, cbufs[0])
        return carry
    ptr, done = lax.fori_loop(0, NSCH // 2, pair, (0, 0))

    trash_l = jnp.full((16,), TRASH, jnp.int32)
    zero_i = jnp.zeros((16,), jnp.int32)
    zero_f = jnp.zeros((16,), jnp.float32)
    for t in range(DB // 16):
        ldsel[pl.ds(ptr + t * 16, 16)] = trash_l
        srcsel[pl.ds(ptr + t * 16, 16)] = zero_i
        wsel[pl.ds(ptr + t * 16, 16)] = zero_f
    _, done = flush((ptr, done))

    cbuf[...] = jnp.full((16,), done, jnp.int32)
    pltpu.sync_copy(cbuf, cnt_hbm.at[pl.ds(wid * 16, 16)])


@functools.partial(
    pl.kernel,
    out_type=(
        jax.ShapeDtypeStruct((NW * CAP,), jnp.int32),
        jax.ShapeDtypeStruct((NW * CAP,), jnp.int32),
        jax.ShapeDtypeStruct((NW * CAP,), jnp.float32),
        jax.ShapeDtypeStruct((NW * 16,), jnp.int32),
    ),
    mesh=plsc.VectorSubcoreMesh(**_SC_MESH),
    scratch_types=[
        pltpu.VMEM((SCH,), jnp.int32),
        pltpu.VMEM((SCH,), jnp.int32),
        pltpu.VMEM((SCH,), jnp.float32),
        pltpu.VMEM((SCH,), jnp.int32),
        pltpu.VMEM((SCH,), jnp.int32),
        pltpu.VMEM((SCH,), jnp.float32),
        pltpu.VMEM((DB + DB + 16,), jnp.int32),
        pltpu.VMEM((DB + DB + 16,), jnp.int32),
        pltpu.VMEM((DB + DB + 16,), jnp.float32),
        pltpu.VMEM((16,), jnp.int32),
        pltpu.SemaphoreType.DMA,
        pltpu.SemaphoreType.DMA,
    ],
)
def _edge_prep(*args):
    _prep_body(*args)


def _apply_body(h_hbm, ldst_hbm, srcl_hbm, wl_hbm, cnt_hbm, out_hbm,
                lbuf0, sbuf0, wbuf0, rows0, lbuf1, sbuf1, wbuf1, rows1,
                cbuf, acc, seml0, semr0, seml1, semr1):
    c = lax.axis_index("c")
    s = lax.axis_index("s")
    wid = c * NTL + s
    lo = wid * RPT
    lbase = wid * CAP

    pltpu.sync_copy(h_hbm.at[pl.ds(lo, RPT)], acc.at[pl.ds(0, RPT)])

    @pl.when(wid == NW - 1)
    def _():
        pltpu.sync_copy(h_hbm.at[pl.ds(lo + RPT, RLAST - RPT)],
                        acc.at[pl.ds(RPT, RLAST - RPT)])

    pltpu.sync_copy(cnt_hbm.at[pl.ds(wid * 16, 16)], cbuf)
    nb2 = cbuf[...][0] * (DB // DBA)

    bufs = ((lbuf0, sbuf0, wbuf0, rows0, seml0, semr0),
            (lbuf1, sbuf1, wbuf1, rows1, seml1, semr1))

    def lists_refs(b, bs):
        o = lbase + b * DBA
        return ((ldst_hbm.at[pl.ds(o, DBA)], bs[0]),
                (srcl_hbm.at[pl.ds(o, DBA)], bs[1]),
                (wl_hbm.at[pl.ds(o, DBA)], bs[2]))

    def issue_lists(b, bs):
        for sref, dref in lists_refs(b, bs):
            pltpu.async_copy(sref, dref, bs[4])

    def wait_lists(b, bs):
        for sref, dref in lists_refs(b, bs):
            pltpu.make_async_copy(sref, dref, bs[4]).wait()

    def process(b, this, other):
        @pl.when(b + 1 < nb2)
        def _():
            wait_lists(b + 1, other)
            pltpu.async_copy(h_hbm.at[other[1]], other[3], other[5])

        pltpu.make_async_copy(h_hbm.at[this[1]], this[3], this[5]).wait()
        rows, lb, wb = this[3], this[0], this[2]

        def acc16(gg, _):
            wg = wb[pl.ds(gg * 16, 16)]
            lg = lb[pl.ds(gg * 16, 16)]
            for j in range(16):
                r = lg[j]
                w = wg[j]
                e = gg * 16 + j
                vals = [acc[r, pl.ds(k * 16, 16)] + rows[e, pl.ds(k * 16, 16)] * w
                        for k in range(D // 16)]
                for k in range(D // 16):
                    acc[r, pl.ds(k * 16, 16)] = vals[k]
            return 0
        lax.fori_loop(0, DBA // 16, acc16, 0)

        @pl.when(b + 2 < nb2)
        def _():
            issue_lists(b + 2, this)

    for sref, dref in lists_refs(0, bufs[0]):
        pltpu.sync_copy(sref, dref)
    pltpu.async_copy(h_hbm.at[bufs[0][1]], bufs[0][3], bufs[0][5])

    @pl.when(nb2 > 1)
    def _():
        issue_lists(1, bufs[1])

    def pair(i, _):
        process(2 * i, bufs[0], bufs[1])

        @pl.when(2 * i + 1 < nb2)
        def _():
            process(2 * i + 1, bufs[1], bufs[0])
        return 0
    lax.fori_loop(0, (nb2 + 1) // 2, pair, 0)

    pltpu.sync_copy(acc.at[pl.ds(0, RPT)], out_hbm.at[pl.ds(lo, RPT)])

    @pl.when(wid == NW - 1)
    def _():
        pltpu.sync_copy(acc.at[pl.ds(RPT, RLAST - RPT)],
                        out_hbm.at[pl.ds(lo + RPT, RLAST - RPT)])


@functools.partial(
    pl.kernel,
    out_type=jax.ShapeDtypeStruct((N, D), jnp.float32),
    mesh=plsc.VectorSubcoreMesh(**_SC_MESH),
    scratch_types=[
        pltpu.VMEM((DBA,), jnp.int32),
        pltpu.VMEM((DBA,), jnp.int32),
        pltpu.VMEM((DBA,), jnp.float32),
        pltpu.VMEM((DBA, D), jnp.float32),
        pltpu.VMEM((DBA,), jnp.int32),
        pltpu.VMEM((DBA,), jnp.int32),
        pltpu.VMEM((DBA,), jnp.float32),
        pltpu.VMEM((DBA, D), jnp.float32),
        pltpu.VMEM((16,), jnp.int32),
        pltpu.VMEM((ACC_ROWS, D), jnp.float32),
        pltpu.SemaphoreType.DMA,
        pltpu.SemaphoreType.DMA,
        pltpu.SemaphoreType.DMA,
        pltpu.SemaphoreType.DMA,
    ],
)
def _spmm_apply(*args):
    _apply_body(*args)


def _mlp_stats_body(agg_ref, w0_ref, b0_ref, w1_ref, b1_ref, x_ref, stats_ref):
    a = agg_ref[...]
    t = jnp.maximum(jnp.dot(a, w0_ref[...], preferred_element_type=jnp.float32)
                    + b0_ref[...], 0.0)
    y = jnp.dot(t, w1_ref[...], preferred_element_type=jnp.float32) + b1_ref[...]
    y = jnp.where(y > 0, y, 0.01 * y)
    x_ref[...] = y

    @pl.when(pl.program_id(0) == 0)
    def _():
        stats_ref[...] = jnp.zeros_like(stats_ref)
    stats_ref[0:1, :] = stats_ref[0:1, :] + jnp.sum(y, axis=0, keepdims=True)
    stats_ref[1:2, :] = stats_ref[1:2, :] + jnp.sum(y * y, axis=0, keepdims=True)


def _mlp_stats(agg, w0, b0, w1, b1):
    return pl.pallas_call(
        _mlp_stats_body,
        grid=(NB,),
        in_specs=[
            pl.BlockSpec((RB, D), lambda i: (i, 0)),
            pl.BlockSpec((D, D), lambda i: (0, 0)),
            pl.BlockSpec((1, D), lambda i: (0, 0)),
            pl.BlockSpec((D, D), lambda i: (0, 0)),
            pl.BlockSpec((1, D), lambda i: (0, 0)),
        ],
        out_specs=[
            pl.BlockSpec((RB, D), lambda i: (i, 0)),
            pl.BlockSpec((8, D), lambda i: (0, 0)),
        ],
        out_shape=[
            jax.ShapeDtypeStruct((N, D), jnp.float32),
            jax.ShapeDtypeStruct((8, D), jnp.float32),
        ],
    )(agg, w0, b0, w1, b1)


def _bn_elin_body(x_ref, stats_ref, gamma_ref, beta_ref, attw_ref, sc_ref,
                  gp1_ref, gp2_ref, h_ref, e_ref, emax_ref):
    mean = stats_ref[0:1, :] * (1.0 / N)
    var = stats_ref[1:2, :] * (1.0 / N) - mean * mean
    inv = lax.rsqrt(var + 1e-5)
    h = gamma_ref[...] * (x_ref[...] - mean) * inv + beta_ref[...]
    h_ref[...] = h
    e = jnp.dot(h, attw_ref[...], preferred_element_type=jnp.float32)
    e = (e + gp1_ref[...] * sc_ref[0:1, 0:1] + gp2_ref[...] * sc_ref[0:1, 1:2]
         + sc_ref[0:1, 2:3])
    e_ref[...] = e

    @pl.when(pl.program_id(0) == 0)
    def _():
        emax_ref[...] = jnp.full_like(emax_ref, -jnp.inf)
    emax_ref[...] = jnp.maximum(emax_ref[...], jnp.max(e))


def _bn_elin(x, stats, gamma, beta, attw, sc, gp1, gp2):
    return pl.pallas_call(
        _bn_elin_body,
        grid=(NB,),
        in_specs=[
            pl.BlockSpec((RB, D), lambda i: (i, 0)),
            pl.BlockSpec((8, D), lambda i: (0, 0)),
            pl.BlockSpec((1, D), lambda i: (0, 0)),
            pl.BlockSpec((1, D), lambda i: (0, 0)),
            pl.BlockSpec((D, 1), lambda i: (0, 0)),
            pl.BlockSpec((1, 128), lambda i: (0, 0)),
            pl.BlockSpec((RB, 1), lambda i: (i, 0)),
            pl.BlockSpec((RB, 1), lambda i: (i, 0)),
        ],
        out_specs=[
            pl.BlockSpec((RB, D), lambda i: (i, 0)),
            pl.BlockSpec((RB, 1), lambda i: (i, 0)),
            pl.BlockSpec((1, 1), lambda i: (0, 0)),
        ],
        out_shape=[
            jax.ShapeDtypeStruct((N, D), jnp.float32),
            jax.ShapeDtypeStruct((N, 1), jnp.float32),
            jax.ShapeDtypeStruct((1, 1), jnp.float32),
        ],
    )(x, stats, gamma, beta, attw, sc, gp1, gp2)


def _elin_body(h_ref, attw_ref, sc_ref, gp1_ref, gp2_ref, e_ref, emax_ref):
    e = jnp.dot(h_ref[...], attw_ref[...], preferred_element_type=jnp.float32)
    e = (e + gp1_ref[...] * sc_ref[0:1, 0:1] + gp2_ref[...] * sc_ref[0:1, 1:2]
         + sc_ref[0:1, 2:3])
    e_ref[...] = e

    @pl.when(pl.program_id(0) == 0)
    def _():
        emax_ref[...] = jnp.full_like(emax_ref, -jnp.inf)
    emax_ref[...] = jnp.maximum(emax_ref[...], jnp.max(e))


def _elin(h, attw, sc, gp1, gp2):
    return pl.pallas_call(
        _elin_body,
        grid=(NB,),
        in_specs=[
            pl.BlockSpec((RB, D), lambda i: (i, 0)),
            pl.BlockSpec((D, 1), lambda i: (0, 0)),
            pl.BlockSpec((1, 128), lambda i: (0, 0)),
            pl.BlockSpec((RB, 1), lambda i: (i, 0)),
            pl.BlockSpec((RB, 1), lambda i: (i, 0)),
        ],
        out_specs=[
            pl.BlockSpec((RB, 1), lambda i: (i, 0)),
            pl.BlockSpec((1, 1), lambda i: (0, 0)),
        ],
        out_shape=[
            jax.ShapeDtypeStruct((N, 1), jnp.float32),
            jax.ShapeDtypeStruct((1, 1), jnp.float32),
        ],
    )(h, attw, sc, gp1, gp2)


def _pool_body(gid_ref, h0_ref, h1_ref, h2_ref, e0_ref, e1_ref, e2_ref,
               m0_ref, m1_ref, m2_ref,
               p0_ref, p1_ref, p2_ref, r0_ref, r1_ref, r2_ref):
    gid = gid_ref[0]
    oh = (gid == lax.broadcasted_iota(jnp.int32, (B, RB), 0)).astype(jnp.float32)

    @pl.when(pl.program_id(0) == 0)
    def _():
        for ref in (p0_ref, p1_ref, p2_ref, r0_ref, r1_ref, r2_ref):
            ref[...] = jnp.zeros_like(ref)

    for h_ref, e_ref, m_ref, p_ref, r_ref in (
            (h0_ref, e0_ref, m0_ref, p0_ref, r0_ref),
            (h1_ref, e1_ref, m1_ref, p1_ref, r1_ref),
            (h2_ref, e2_ref, m2_ref, p2_ref, r2_ref)):
        ee = jnp.exp(e_ref[...] - m_ref[...])
        eh = ee * h_ref[...]
        p_ref[...] = p_ref[...] + jnp.dot(oh, eh, preferred_element_type=jnp.float32)
        eb = jnp.broadcast_to(ee, (RB, 128))
        r_ref[...] = r_ref[...] + jnp.dot(oh, eb, preferred_element_type=jnp.float32)


def _pool(gid3, hs, es, ms):
    blk = lambda shape: pl.BlockSpec(shape, lambda i: (i, 0))
    cst = lambda shape: pl.BlockSpec(shape, lambda i: (0, 0))
    return pl.pallas_call(
        _pool_body,
        grid=(NB,),
        in_specs=[
            pl.BlockSpec((1, 1, RB), lambda i: (i, 0, 0)),
            blk((RB, D)), blk((RB, D)), blk((RB, D)),
            blk((RB, 1)), blk((RB, 1)), blk((RB, 1)),
            cst((1, 1)), cst((1, 1)), cst((1, 1)),
        ],
        out_specs=[cst((B, D)), cst((B, D)), cst((B, D)),
                   cst((B, 128)), cst((B, 128)), cst((B, 128))],
        out_shape=[jax.ShapeDtypeStruct((B, D), jnp.float32)] * 3
                  + [jax.ShapeDtypeStruct((B, 128), jnp.float32)] * 3,
    )(gid3, *hs, *es, *ms)


def _head_body(p0_ref, p1_ref, p2_ref, r0_ref, r1_ref, r2_ref,
               w0_ref, w1_ref, w2_ref, pb_ref,
               score_ref, o0_ref, o1_ref, o2_ref):
    score = jnp.zeros((B, OUT), jnp.float32)
    for i, (p_ref, r_ref, w_ref, o_ref) in enumerate(
            ((p0_ref, r0_ref, w0_ref, o0_ref),
             (p1_ref, r1_ref, w1_ref, o1_ref),
             (p2_ref, r2_ref, w2_ref, o2_ref))):
        pooled = p_ref[...] / (r_ref[:, 0:1] + 1e-10)
        o_ref[...] = pooled
        score = score + jnp.dot(pooled, w_ref[...],
                                preferred_element_type=jnp.float32) \
            + pb_ref[i:i + 1, :]
    score_ref[...] = score


def _head(praws, rsums, predws, predb):
    full = lambda shape: pl.BlockSpec(shape, lambda: (0, 0))
    return pl.pallas_call(
        _head_body,
        in_specs=[full((B, D))] * 3 + [full((B, 128))] * 3
                 + [full((D, OUT))] * 3 + [full((3, OUT))],
        out_specs=[full((B, OUT))] + [full((B, D))] * 3,
        out_shape=[jax.ShapeDtypeStruct((B, OUT), jnp.float32)]
                  + [jax.ShapeDtypeStruct((B, D), jnp.float32)] * 3,
    )(*praws, *rsums, *predws, predb)


def kernel(node_ids, pos_enc, edge_index, edge_weights, graph_ids, elem_gp1,
           elem_gp2, word_emb, pos, gnn_W0, gnn_b0, gnn_W1, gnn_b1, bn_gamma,
           bn_beta, att_W, att_b, pred_W, pred_b):
    src = edge_index[0]
    dst = edge_index[1]
    gp1 = elem_gp1.reshape(N, 1)
    gp2 = elem_gp2.reshape(N, 1)
    gid3 = graph_ids.reshape(NB, 1, RB)

    def att_params(l):
        attw = att_W[l, :D, :]
        sc = jnp.zeros((1, 128), jnp.float32)
        sc = sc.at[0, 0].set(att_W[l, D, 0])
        sc = sc.at[0, 1].set(att_W[l, D + 1, 0])
        sc = sc.at[0, 2].set(att_b[l, 0])
        return attw, sc

    pos16 = jnp.broadcast_to(pos[0:1], (16,))
    h = _embed(node_ids, pos_enc, word_emb, pos16)
    elist_ldst, elist_src, elist_w, elist_cnt = _edge_prep(src, dst,
                                                           edge_weights)

    attw0, sc0 = att_params(0)
    e0, m0 = _elin(h, attw0, sc0, gp1, gp2)

    hs, es, ms = [h], [e0], [m0]
    for l in range(2):
        agg = _spmm_apply(h, elist_ldst, elist_src, elist_w, elist_cnt)

        x, stats = _mlp_stats(agg, gnn_W0[l], gnn_b0[l].reshape(1, D),
                              gnn_W1[l], gnn_b1[l].reshape(1, D))
        attw, sc = att_params(l + 1)
        h, e, m = _bn_elin(x, stats, bn_gamma[l].reshape(1, D),
                           bn_beta[l].reshape(1, D), attw, sc, gp1, gp2)
        hs.append(h); es.append(e); ms.append(m)

    p0, p1, p2, r0, r1, r2 = _pool(gid3, hs, es, ms)
    score, o0, o1, o2 = _head((p0, p1, p2), (r0, r1, r2),
                              (pred_W[0], pred_W[1], pred_W[2]), pred_b)
    return (score, o0, o1, o2)


# speedup vs baseline: 2.1686x; 1.1333x over previous
import functools

import jax
import jax.numpy as jnp
from jax import lax
from jax.experimental import pallas as pl
from jax.experimental.pallas import tpu as pltpu
from jax.experimental.pallas import tpu_sc as plsc

N = 10000
E = 160000
D = 256
B = 16
OUT = 16
RB = 1000
NB = N // RB

NSC = 2
NTL = 16
NW = NSC * NTL
RPT = 312
RLAST = N - RPT * (NW - 1)
TRASH = 328
ACC_ROWS = 336
DB = 128
DBA = 64
CAP = E + DB
SCH = 1600
NSCH = E // SCH
NGRP = SCH // 32
ERC = 80
_SC_MESH = dict(core_axis_name="c", subcore_axis_name="s",
                num_cores=NSC, num_subcores=NTL)


def _embed_body(ids_hbm, pe_hbm, emb_hbm, p16_hbm, out_hbm,
                idxv, rows, pev, p16v, sem):
    c = lax.axis_index("c")
    s = lax.axis_index("s")
    wid = s * NSC + c
    start = jnp.minimum(wid * (4 * ERC), N - 4 * ERC)
    pltpu.sync_copy(p16_hbm, p16v)
    p0 = p16v[...]
    for j in range(4):
        o = start + j * ERC
        pltpu.sync_copy(ids_hbm.at[pl.ds(o, ERC)], idxv)
        pltpu.async_copy(emb_hbm.at[idxv], rows, sem).wait()
        pltpu.sync_copy(pe_hbm.at[pl.ds(o, ERC)], pev)

        def addrow(r, _):
            for k in range(D // 16):
                sl = pl.ds(k * 16, 16)
                rows[r, sl] = rows[r, sl] + p0 * pev[r, sl]
            return 0
        lax.fori_loop(0, ERC, addrow, 0)
        pltpu.sync_copy(rows, out_hbm.at[pl.ds(o, ERC)])


@functools.partial(
    pl.kernel,
    out_type=jax.ShapeDtypeStruct((N, D), jnp.float32),
    mesh=plsc.VectorSubcoreMesh(**_SC_MESH),
    scratch_types=[
        pltpu.VMEM((ERC,), jnp.int32),
        pltpu.VMEM((ERC, D), jnp.float32),
        pltpu.VMEM((ERC, D), jnp.float32),
        pltpu.VMEM((16,), jnp.float32),
        pltpu.SemaphoreType.DMA,
    ],
)
def _embed(*args):
    _embed_body(*args)


_GDN = lax.GatherDimensionNumbers(
    offset_dims=(), collapsed_slice_dims=(0,), start_index_map=(0,))


def _dg(vec, idx):
    return lax.gather(vec, idx[:, None], _GDN, (1,),
                      mode=lax.GatherScatterMode.PROMISE_IN_BOUNDS)


def _prep_body(src_hbm, dst_hbm, w_hbm,
               ldst_hbm, srcl_hbm, wl_hbm, cnt_hbm,
               dstb, srcb, wch, dstb1, srcb1, wch1, ldsel, srcsel, wsel, cbuf,
               semc0, semc1):
    c = lax.axis_index("c")
    s = lax.axis_index("s")
    wid = c * NTL + s
    lo = wid * RPT
    hi = lo + jnp.where(wid == NW - 1, RLAST, RPT)
    lbase = wid * CAP
    lane = lax.iota(jnp.int32, 16)

    def flush(carry):
        ptr, done = carry
        o = lbase + done * DB
        pltpu.sync_copy(ldsel.at[pl.ds(0, DB)], ldst_hbm.at[pl.ds(o, DB)])
        pltpu.sync_copy(srcsel.at[pl.ds(0, DB)], srcl_hbm.at[pl.ds(o, DB)])
        pltpu.sync_copy(wsel.at[pl.ds(0, DB)], wl_hbm.at[pl.ds(o, DB)])
        ldsel[pl.ds(0, 16)] = ldsel[pl.ds(DB, 16)]
        srcsel[pl.ds(0, 16)] = srcsel[pl.ds(DB, 16)]
        wsel[pl.ds(0, 16)] = wsel[pl.ds(DB, 16)]
        return ptr - DB, done + 1

    shidx = [jnp.maximum(lane - k, 0) for k in (1, 2, 4, 8)]
    shmask = [lane >= k for k in (1, 2, 4, 8)]
    zero16 = jnp.zeros((16,), jnp.int32)

    def scan_chunk(ch, carry, db, sb, wb):
        def prefix(m):
            x = jnp.where(m, 1, 0)
            for k in range(4):
                x = x + jnp.where(shmask[k], _dg(x, shidx[k]), 0)
            return x

        def mksel(v, x, cnt, sl16, sb, wb):
            def sel(carry):
                ptr, done = carry
                tgt = lane + 1
                pos = zero16
                for st in (8, 4, 2, 1):
                    cand = pos + st
                    xv = _dg(x, cand - 1)
                    pos = jnp.where(xv < tgt, cand, pos)
                srci = jnp.minimum(pos, 15)
                ldsel[pl.ds(ptr, 16)] = _dg(v, srci) - lo
                srcsel[pl.ds(ptr, 16)] = _dg(sb[sl16], srci)
                wsel[pl.ds(ptr, 16)] = _dg(wb[sl16], srci)
                ptr = ptr + cnt
                return lax.cond(ptr >= DB, flush, lambda cc: cc, (ptr, done))
            return sel

        def grp(g, carry):
            sl_a = pl.ds(g * 32, 16)
            sl_b = pl.ds(g * 32 + 16, 16)
            va = db[sl_a]
            vb = db[sl_b]
            ma = (va >= lo) & (va < hi)
            mb = (vb >= lo) & (vb < hi)
            xa = prefix(ma)
            xb = prefix(mb)
            ca = xa[15]
            cb = xb[15]
            carry = lax.cond(ca > 0, mksel(va, xa, ca, sl_a, sb, wb),
                             lambda cc: cc, carry)
            carry = lax.cond(cb > 0, mksel(vb, xb, cb, sl_b, sb, wb),
                             lambda cc: cc, carry)
            return carry
        return lax.fori_loop(0, NGRP, grp, carry)

    cbufs = ((dstb, srcb, wch, semc0), (dstb1, srcb1, wch1, semc1))

    def chunk_refs(ch, cb):
        csl = pl.ds(ch * SCH, SCH)
        return ((dst_hbm.at[csl], cb[0]), (src_hbm.at[csl], cb[1]),
                (w_hbm.at[csl], cb[2]))

    def issue_chunk(ch, cb):
        for sref, dref in chunk_refs(ch, cb):
            pltpu.async_copy(sref, dref, cb[3])

    def wait_chunk(ch, cb):
        for sref, dref in chunk_refs(ch, cb):
            pltpu.make_async_copy(sref, dref, cb[3]).wait()

    for sref, dref in chunk_refs(0, cbufs[0]):
        pltpu.sync_copy(sref, dref)
    issue_chunk(1, cbufs[1])

    def pair(i, carry):
        carry = scan_chunk(2 * i, carry, cbufs[0][0], cbufs[0][1], cbufs[0][2])

        @pl.when(i < NSCH // 2 - 1)
        def _():
            issue_chunk(2 * i + 2, cbufs[0])
        wait_chunk(2 * i + 1, cbufs[1])
        carry = scan_chunk(2 * i + 1, carry,
                           cbufs[1][0], cbufs[1][1], cbufs[1][2])

        @pl.when(i < NSCH // 2 - 1)
        def _():
            issue_chunk(2 * i + 3, cbufs[1])
            wait_chunk(2 * i + 2, cbufs[0])
        return carry
    ptr, done = lax.fori_loop(0, NSCH // 2, pair, (0, 0))

    trash_l = jnp.full((16,), TRASH, jnp.int32)
    zero_i = jnp.zeros((16,), jnp.int32)
    zero_f = jnp.zeros((16,), jnp.float32)
    for t in range(DB // 16):
        ldsel[pl.ds(ptr + t * 16, 16)] = trash_l
        srcsel[pl.ds(ptr + t * 16, 16)] = zero_i
        wsel[pl.ds(ptr + t * 16, 16)] = zero_f
    _, done = flush((ptr, done))

    cbuf[...] = jnp.full((16,), done, jnp.int32)
    pltpu.sync_copy(cbuf, cnt_hbm.at[pl.ds(wid * 16, 16)])


@functools.partial(
    pl.kernel,
    out_type=(
        jax.ShapeDtypeStruct((NW * CAP,), jnp.int32),
        jax.ShapeDtypeStruct((NW * CAP,), jnp.int32),
        jax.ShapeDtypeStruct((NW * CAP,), jnp.float32),
        jax.ShapeDtypeStruct((NW * 16,), jnp.int32),
    ),
    mesh=plsc.VectorSubcoreMesh(**_SC_MESH),
    scratch_types=[
        pltpu.VMEM((SCH,), jnp.int32),
        pltpu.VMEM((SCH,), jnp.int32),
        pltpu.VMEM((SCH,), jnp.float32),
        pltpu.VMEM((SCH,), jnp.int32),
        pltpu.VMEM((SCH,), jnp.int32),
        pltpu.VMEM((SCH,), jnp.float32),
        pltpu.VMEM((DB + DB + 16,), jnp.int32),
        pltpu.VMEM((DB + DB + 16,), jnp.int32),
        pltpu.VMEM((DB + DB + 16,), jnp.float32),
        pltpu.VMEM((16,), jnp.int32),
        pltpu.SemaphoreType.DMA,
        pltpu.SemaphoreType.DMA,
    ],
)
def _edge_prep(*args):
    _prep_body(*args)


def _apply_body(h_hbm, ldst_hbm, srcl_hbm, wl_hbm, cnt_hbm, out_hbm,
                lbuf0, sbuf0, wbuf0, rows0, lbuf1, sbuf1, wbuf1, rows1,
                cbuf, acc, seml0, semr0, seml1, semr1):
    c = lax.axis_index("c")
    s = lax.axis_index("s")
    wid = c * NTL + s
    lo = wid * RPT
    lbase = wid * CAP

    pltpu.sync_copy(h_hbm.at[pl.ds(lo, RPT)], acc.at[pl.ds(0, RPT)])

    @pl.when(wid == NW - 1)
    def _():
        pltpu.sync_copy(h_hbm.at[pl.ds(lo + RPT, RLAST - RPT)],
                        acc.at[pl.ds(RPT, RLAST - RPT)])

    pltpu.sync_copy(cnt_hbm.at[pl.ds(wid * 16, 16)], cbuf)
    nb2 = cbuf[...][0] * (DB // DBA)

    bufs = ((lbuf0, sbuf0, wbuf0, rows0, seml0, semr0),
            (lbuf1, sbuf1, wbuf1, rows1, seml1, semr1))

    def lists_refs(b, bs):
        o = lbase + b * DBA
        return ((ldst_hbm.at[pl.ds(o, DBA)], bs[0]),
                (srcl_hbm.at[pl.ds(o, DBA)], bs[1]),
                (wl_hbm.at[pl.ds(o, DBA)], bs[2]))

    def issue_lists(b, bs):
        for sref, dref in lists_refs(b, bs):
            pltpu.async_copy(sref, dref, bs[4])

    def wait_lists(b, bs):
        for sref, dref in lists_refs(b, bs):
            pltpu.make_async_copy(sref, dref, bs[4]).wait()

    def process(b, this, other):
        @pl.when(b + 1 < nb2)
        def _():
            wait_lists(b + 1, other)
            pltpu.async_copy(h_hbm.at[other[1]], other[3], other[5])

        pltpu.make_async_copy(h_hbm.at[this[1]], this[3], this[5]).wait()
        rows, lb, wb = this[3], this[0], this[2]

        def acc16(gg, _):
            wg = wb[pl.ds(gg * 16, 16)]
            lg = lb[pl.ds(gg * 16, 16)]
            for j in range(16):
                r = lg[j]
                w = wg[j]
                e = gg * 16 + j
                vals = [acc[r, pl.ds(k * 16, 16)] + rows[e, pl.ds(k * 16, 16)] * w
                        for k in range(D // 16)]
                for k in range(D // 16):
                    acc[r, pl.ds(k * 16, 16)] = vals[k]
            return 0
        lax.fori_loop(0, DBA // 16, acc16, 0)

        @pl.when(b + 2 < nb2)
        def _():
            issue_lists(b + 2, this)

    for sref, dref in lists_refs(0, bufs[0]):
        pltpu.sync_copy(sref, dref)
    pltpu.async_copy(h_hbm.at[bufs[0][1]], bufs[0][3], bufs[0][5])

    @pl.when(nb2 > 1)
    def _():
        issue_lists(1, bufs[1])

    def pair(i, _):
        process(2 * i, bufs[0], bufs[1])

        @pl.when(2 * i + 1 < nb2)
        def _():
            process(2 * i + 1, bufs[1], bufs[0])
        return 0
    lax.fori_loop(0, (nb2 + 1) // 2, pair, 0)

    pltpu.sync_copy(acc.at[pl.ds(0, RPT)], out_hbm.at[pl.ds(lo, RPT)])

    @pl.when(wid == NW - 1)
    def _():
        pltpu.sync_copy(acc.at[pl.ds(RPT, RLAST - RPT)],
                        out_hbm.at[pl.ds(lo + RPT, RLAST - RPT)])


@functools.partial(
    pl.kernel,
    out_type=jax.ShapeDtypeStruct((N, D), jnp.float32),
    mesh=plsc.VectorSubcoreMesh(**_SC_MESH),
    scratch_types=[
        pltpu.VMEM((DBA,), jnp.int32),
        pltpu.VMEM((DBA,), jnp.int32),
        pltpu.VMEM((DBA,), jnp.float32),
        pltpu.VMEM((DBA, D), jnp.float32),
        pltpu.VMEM((DBA,), jnp.int32),
        pltpu.VMEM((DBA,), jnp.int32),
        pltpu.VMEM((DBA,), jnp.float32),
        pltpu.VMEM((DBA, D), jnp.float32),
        pltpu.VMEM((16,), jnp.int32),
        pltpu.VMEM((ACC_ROWS, D), jnp.float32),
        pltpu.SemaphoreType.DMA,
        pltpu.SemaphoreType.DMA,
        pltpu.SemaphoreType.DMA,
        pltpu.SemaphoreType.DMA,
    ],
)
def _spmm_apply(*args):
    _apply_body(*args)


def _mlp_stats_body(agg_ref, w0_ref, b0_ref, w1_ref, b1_ref, x_ref, stats_ref):
    a = agg_ref[...]
    t = jnp.maximum(jnp.dot(a, w0_ref[...], preferred_element_type=jnp.float32)
                    + b0_ref[...], 0.0)
    y = jnp.dot(t, w1_ref[...], preferred_element_type=jnp.float32) + b1_ref[...]
    y = jnp.where(y > 0, y, 0.01 * y)
    x_ref[...] = y

    @pl.when(pl.program_id(0) == 0)
    def _():
        stats_ref[...] = jnp.zeros_like(stats_ref)
    stats_ref[0:1, :] = stats_ref[0:1, :] + jnp.sum(y, axis=0, keepdims=True)
    stats_ref[1:2, :] = stats_ref[1:2, :] + jnp.sum(y * y, axis=0, keepdims=True)


def _mlp_stats(agg, w0, b0, w1, b1):
    return pl.pallas_call(
        _mlp_stats_body,
        grid=(NB,),
        in_specs=[
            pl.BlockSpec((RB, D), lambda i: (i, 0)),
            pl.BlockSpec((D, D), lambda i: (0, 0)),
            pl.BlockSpec((1, D), lambda i: (0, 0)),
            pl.BlockSpec((D, D), lambda i: (0, 0)),
            pl.BlockSpec((1, D), lambda i: (0, 0)),
        ],
        out_specs=[
            pl.BlockSpec((RB, D), lambda i: (i, 0)),
            pl.BlockSpec((8, D), lambda i: (0, 0)),
        ],
        out_shape=[
            jax.ShapeDtypeStruct((N, D), jnp.float32),
            jax.ShapeDtypeStruct((8, D), jnp.float32),
        ],
    )(agg, w0, b0, w1, b1)


def _bn_elin_body(x_ref, stats_ref, gamma_ref, beta_ref, attw_ref, sc_ref,
                  gp1_ref, gp2_ref, h_ref, e_ref, emax_ref):
    mean = stats_ref[0:1, :] * (1.0 / N)
    var = stats_ref[1:2, :] * (1.0 / N) - mean * mean
    inv = lax.rsqrt(var + 1e-5)
    h = gamma_ref[...] * (x_ref[...] - mean) * inv + beta_ref[...]
    h_ref[...] = h
    e = jnp.dot(h, attw_ref[...], preferred_element_type=jnp.float32)
    e = (e + gp1_ref[...] * sc_ref[0:1, 0:1] + gp2_ref[...] * sc_ref[0:1, 1:2]
         + sc_ref[0:1, 2:3])
    e_ref[...] = e

    @pl.when(pl.program_id(0) == 0)
    def _():
        emax_ref[...] = jnp.full_like(emax_ref, -jnp.inf)
    emax_ref[...] = jnp.maximum(emax_ref[...], jnp.max(e))


def _bn_elin(x, stats, gamma, beta, attw, sc, gp1, gp2):
    return pl.pallas_call(
        _bn_elin_body,
        grid=(NB,),
        in_specs=[
            pl.BlockSpec((RB, D), lambda i: (i, 0)),
            pl.BlockSpec((8, D), lambda i: (0, 0)),
            pl.BlockSpec((1, D), lambda i: (0, 0)),
            pl.BlockSpec((1, D), lambda i: (0, 0)),
            pl.BlockSpec((D, 1), lambda i: (0, 0)),
            pl.BlockSpec((1, 128), lambda i: (0, 0)),
            pl.BlockSpec((RB, 1), lambda i: (i, 0)),
            pl.BlockSpec((RB, 1), lambda i: (i, 0)),
        ],
        out_specs=[
            pl.BlockSpec((RB, D), lambda i: (i, 0)),
            pl.BlockSpec((RB, 1), lambda i: (i, 0)),
            pl.BlockSpec((1, 1), lambda i: (0, 0)),
        ],
        out_shape=[
            jax.ShapeDtypeStruct((N, D), jnp.float32),
            jax.ShapeDtypeStruct((N, 1), jnp.float32),
            jax.ShapeDtypeStruct((1, 1), jnp.float32),
        ],
    )(x, stats, gamma, beta, attw, sc, gp1, gp2)


def _elin_body(h_ref, attw_ref, sc_ref, gp1_ref, gp2_ref, e_ref, emax_ref):
    e = jnp.dot(h_ref[...], attw_ref[...], preferred_element_type=jnp.float32)
    e = (e + gp1_ref[...] * sc_ref[0:1, 0:1] + gp2_ref[...] * sc_ref[0:1, 1:2]
         + sc_ref[0:1, 2:3])
    e_ref[...] = e

    @pl.when(pl.program_id(0) == 0)
    def _():
        emax_ref[...] = jnp.full_like(emax_ref, -jnp.inf)
    emax_ref[...] = jnp.maximum(emax_ref[...], jnp.max(e))


def _elin(h, attw, sc, gp1, gp2):
    return pl.pallas_call(
        _elin_body,
        grid=(NB,),
        in_specs=[
            pl.BlockSpec((RB, D), lambda i: (i, 0)),
            pl.BlockSpec((D, 1), lambda i: (0, 0)),
            pl.BlockSpec((1, 128), lambda i: (0, 0)),
            pl.BlockSpec((RB, 1), lambda i: (i, 0)),
            pl.BlockSpec((RB, 1), lambda i: (i, 0)),
        ],
        out_specs=[
            pl.BlockSpec((RB, 1), lambda i: (i, 0)),
            pl.BlockSpec((1, 1), lambda i: (0, 0)),
        ],
        out_shape=[
            jax.ShapeDtypeStruct((N, 1), jnp.float32),
            jax.ShapeDtypeStruct((1, 1), jnp.float32),
        ],
    )(h, attw, sc, gp1, gp2)


def _pool_body(gid_ref, h0_ref, h1_ref, h2_ref, e0_ref, e1_ref, e2_ref,
               m0_ref, m1_ref, m2_ref,
               p0_ref, p1_ref, p2_ref, r0_ref, r1_ref, r2_ref):
    gid = gid_ref[0]
    oh = (gid == lax.broadcasted_iota(jnp.int32, (B, RB), 0)).astype(jnp.float32)

    @pl.when(pl.program_id(0) == 0)
    def _():
        for ref in (p0_ref, p1_ref, p2_ref, r0_ref, r1_ref, r2_ref):
            ref[...] = jnp.zeros_like(ref)

    for h_ref, e_ref, m_ref, p_ref, r_ref in (
            (h0_ref, e0_ref, m0_ref, p0_ref, r0_ref),
            (h1_ref, e1_ref, m1_ref, p1_ref, r1_ref),
            (h2_ref, e2_ref, m2_ref, p2_ref, r2_ref)):
        ee = jnp.exp(e_ref[...] - m_ref[...])
        eh = ee * h_ref[...]
        p_ref[...] = p_ref[...] + jnp.dot(oh, eh, preferred_element_type=jnp.float32)
        eb = jnp.broadcast_to(ee, (RB, 128))
        r_ref[...] = r_ref[...] + jnp.dot(oh, eb, preferred_element_type=jnp.float32)


def _pool(gid3, hs, es, ms):
    blk = lambda shape: pl.BlockSpec(shape, lambda i: (i, 0))
    cst = lambda shape: pl.BlockSpec(shape, lambda i: (0, 0))
    return pl.pallas_call(
        _pool_body,
        grid=(NB,),
        in_specs=[
            pl.BlockSpec((1, 1, RB), lambda i: (i, 0, 0)),
            blk((RB, D)), blk((RB, D)), blk((RB, D)),
            blk((RB, 1)), blk((RB, 1)), blk((RB, 1)),
            cst((1, 1)), cst((1, 1)), cst((1, 1)),
        ],
        out_specs=[cst((B, D)), cst((B, D)), cst((B, D)),
                   cst((B, 128)), cst((B, 128)), cst((B, 128))],
        out_shape=[jax.ShapeDtypeStruct((B, D), jnp.float32)] * 3
                  + [jax.ShapeDtypeStruct((B, 128), jnp.float32)] * 3,
    )(gid3, *hs, *es, *ms)


def _head_body(p0_ref, p1_ref, p2_ref, r0_ref, r1_ref, r2_ref,
               w0_ref, w1_ref, w2_ref, pb_ref,
               score_ref, o0_ref, o1_ref, o2_ref):
    score = jnp.zeros((B, OUT), jnp.float32)
    for i, (p_ref, r_ref, w_ref, o_ref) in enumerate(
            ((p0_ref, r0_ref, w0_ref, o0_ref),
             (p1_ref, r1_ref, w1_ref, o1_ref),
             (p2_ref, r2_ref, w2_ref, o2_ref))):
        pooled = p_ref[...] / (r_ref[:, 0:1] + 1e-10)
        o_ref[...] = pooled
        score = score + jnp.dot(pooled, w_ref[...],
                                preferred_element_type=jnp.float32) \
            + pb_ref[i:i + 1, :]
    score_ref[...] = score


def _head(praws, rsums, predws, predb):
    full = lambda shape: pl.BlockSpec(shape, lambda: (0, 0))
    return pl.pallas_call(
        _head_body,
        in_specs=[full((B, D))] * 3 + [full((B, 128))] * 3
                 + [full((D, OUT))] * 3 + [full((3, OUT))],
        out_specs=[full((B, OUT))] + [full((B, D))] * 3,
        out_shape=[jax.ShapeDtypeStruct((B, OUT), jnp.float32)]
                  + [jax.ShapeDtypeStruct((B, D), jnp.float32)] * 3,
    )(*praws, *rsums, *predws, predb)


def kernel(node_ids, pos_enc, edge_index, edge_weights, graph_ids, elem_gp1,
           elem_gp2, word_emb, pos, gnn_W0, gnn_b0, gnn_W1, gnn_b1, bn_gamma,
           bn_beta, att_W, att_b, pred_W, pred_b):
    src = edge_index[0]
    dst = edge_index[1]
    gp1 = elem_gp1.reshape(N, 1)
    gp2 = elem_gp2.reshape(N, 1)
    gid3 = graph_ids.reshape(NB, 1, RB)

    def att_params(l):
        attw = att_W[l, :D, :]
        sc = jnp.zeros((1, 128), jnp.float32)
        sc = sc.at[0, 0].set(att_W[l, D, 0])
        sc = sc.at[0, 1].set(att_W[l, D + 1, 0])
        sc = sc.at[0, 2].set(att_b[l, 0])
        return attw, sc

    pos16 = jnp.broadcast_to(pos[0:1], (16,))
    h = _embed(node_ids, pos_enc, word_emb, pos16)
    elist_ldst, elist_src, elist_w, elist_cnt = _edge_prep(src, dst,
                                                           edge_weights)

    attw0, sc0 = att_params(0)
    e0, m0 = _elin(h, attw0, sc0, gp1, gp2)

    hs, es, ms = [h], [e0], [m0]
    for l in range(2):
        agg = _spmm_apply(h, elist_ldst, elist_src, elist_w, elist_cnt)

        x, stats = _mlp_stats(agg, gnn_W0[l], gnn_b0[l].reshape(1, D),
                              gnn_W1[l], gnn_b1[l].reshape(1, D))
        attw, sc = att_params(l + 1)
        h, e, m = _bn_elin(x, stats, bn_gamma[l].reshape(1, D),
                           bn_beta[l].reshape(1, D), attw, sc, gp1, gp2)
        hs.append(h); es.append(e); ms.append(m)

    p0, p1, p2, r0, r1, r2 = _pool(gid3, hs, es, ms)
    score, o0, o1, o2 = _head((p0, p1, p2), (r0, r1, r2),
                              (pred_W[0], pred_W[1], pred_W[2]), pred_b)
    return (score, o0, o1, o2)


# speedup vs baseline: 2.3312x; 1.0750x over previous
import functools

import jax
import jax.numpy as jnp
from jax import lax
from jax.experimental import pallas as pl
from jax.experimental.pallas import tpu as pltpu
from jax.experimental.pallas import tpu_sc as plsc

N = 10000
E = 160000
D = 256
B = 16
OUT = 16
RB = 1000
NB = N // RB

NSC = 2
NTL = 16
NW = NSC * NTL
RPT = 312
RLAST = N - RPT * (NW - 1)
TRASH = 328
ACC_ROWS = 336
DB = 128
DBA = 64
CAP = E + DB
SCH = 1600
NSCH = E // SCH
NGRP = SCH // 64
ERC = 80
_SC_MESH = dict(core_axis_name="c", subcore_axis_name="s",
                num_cores=NSC, num_subcores=NTL)


def _embed_body(ids_hbm, pe_hbm, emb_hbm, p16_hbm, out_hbm,
                idxv, rows, pev, p16v, sem):
    c = lax.axis_index("c")
    s = lax.axis_index("s")
    wid = s * NSC + c
    start = jnp.minimum(wid * (4 * ERC), N - 4 * ERC)
    pltpu.sync_copy(p16_hbm, p16v)
    p0 = p16v[...]
    for j in range(4):
        o = start + j * ERC
        pltpu.sync_copy(ids_hbm.at[pl.ds(o, ERC)], idxv)
        pltpu.async_copy(emb_hbm.at[idxv], rows, sem).wait()
        pltpu.sync_copy(pe_hbm.at[pl.ds(o, ERC)], pev)

        def addrow(r, _):
            for k in range(D // 16):
                sl = pl.ds(k * 16, 16)
                rows[r, sl] = rows[r, sl] + p0 * pev[r, sl]
            return 0
        lax.fori_loop(0, ERC, addrow, 0)
        pltpu.sync_copy(rows, out_hbm.at[pl.ds(o, ERC)])


@functools.partial(
    pl.kernel,
    out_type=jax.ShapeDtypeStruct((N, D), jnp.float32),
    mesh=plsc.VectorSubcoreMesh(**_SC_MESH),
    scratch_types=[
        pltpu.VMEM((ERC,), jnp.int32),
        pltpu.VMEM((ERC, D), jnp.float32),
        pltpu.VMEM((ERC, D), jnp.float32),
        pltpu.VMEM((16,), jnp.float32),
        pltpu.SemaphoreType.DMA,
    ],
)
def _embed(*args):
    _embed_body(*args)


_GDN = lax.GatherDimensionNumbers(
    offset_dims=(), collapsed_slice_dims=(0,), start_index_map=(0,))


def _dg(vec, idx):
    return lax.gather(vec, idx[:, None], _GDN, (1,),
                      mode=lax.GatherScatterMode.PROMISE_IN_BOUNDS)


def _prep_body(src_hbm, dst_hbm, w_hbm,
               ldst_hbm, srcl_hbm, wl_hbm, cnt_hbm,
               dstb, srcb, wch, dstb1, srcb1, wch1, ldsel, srcsel, wsel, cbuf,
               semc0, semc1):
    c = lax.axis_index("c")
    s = lax.axis_index("s")
    wid = c * NTL + s
    lo = wid * RPT
    hi = lo + jnp.where(wid == NW - 1, RLAST, RPT)
    lbase = wid * CAP
    lane = lax.iota(jnp.int32, 16)

    def flush(carry):
        ptr, done = carry
        o = lbase + done * DB
        pltpu.sync_copy(ldsel.at[pl.ds(0, DB)], ldst_hbm.at[pl.ds(o, DB)])
        pltpu.sync_copy(srcsel.at[pl.ds(0, DB)], srcl_hbm.at[pl.ds(o, DB)])
        pltpu.sync_copy(wsel.at[pl.ds(0, DB)], wl_hbm.at[pl.ds(o, DB)])
        ldsel[pl.ds(0, 16)] = ldsel[pl.ds(DB, 16)]
        srcsel[pl.ds(0, 16)] = srcsel[pl.ds(DB, 16)]
        wsel[pl.ds(0, 16)] = wsel[pl.ds(DB, 16)]
        return ptr - DB, done + 1

    shidx = [jnp.maximum(lane - k, 0) for k in (1, 2, 4, 8)]
    shmask = [lane >= k for k in (1, 2, 4, 8)]
    zero16 = jnp.zeros((16,), jnp.int32)

    def scan_chunk(ch, carry, db, sb, wb):
        def prefix(m):
            x = jnp.where(m, 1, 0)
            for k in range(4):
                x = x + jnp.where(shmask[k], _dg(x, shidx[k]), 0)
            return x

        def mksel(v, x, cnt, sl16, sb, wb):
            def sel(carry):
                ptr, done = carry
                tgt = lane + 1
                pos = zero16
                for st in (8, 4, 2, 1):
                    cand = pos + st
                    xv = _dg(x, cand - 1)
                    pos = jnp.where(xv < tgt, cand, pos)
                srci = jnp.minimum(pos, 15)
                ldsel[pl.ds(ptr, 16)] = _dg(v, srci) - lo
                srcsel[pl.ds(ptr, 16)] = _dg(sb[sl16], srci)
                wsel[pl.ds(ptr, 16)] = _dg(wb[sl16], srci)
                ptr = ptr + cnt
                return lax.cond(ptr >= DB, flush, lambda cc: cc, (ptr, done))
            return sel

        def grp(g, carry):
            sls = [pl.ds(g * 64 + 16 * t, 16) for t in range(4)]
            vs = [db[sl] for sl in sls]
            ms = [(v >= lo) & (v < hi) for v in vs]
            xs = [prefix(m) for m in ms]
            cs = [x[15] for x in xs]
            for t in range(4):
                carry = lax.cond(cs[t] > 0,
                                 mksel(vs[t], xs[t], cs[t], sls[t], sb, wb),
                                 lambda cc: cc, carry)
            return carry
        return lax.fori_loop(0, NGRP, grp, carry)

    cbufs = ((dstb, srcb, wch, semc0), (dstb1, srcb1, wch1, semc1))

    def chunk_refs(ch, cb):
        csl = pl.ds(ch * SCH, SCH)
        return ((dst_hbm.at[csl], cb[0]), (src_hbm.at[csl], cb[1]),
                (w_hbm.at[csl], cb[2]))

    def issue_chunk(ch, cb):
        for sref, dref in chunk_refs(ch, cb):
            pltpu.async_copy(sref, dref, cb[3])

    def wait_chunk(ch, cb):
        for sref, dref in chunk_refs(ch, cb):
            pltpu.make_async_copy(sref, dref, cb[3]).wait()

    for sref, dref in chunk_refs(0, cbufs[0]):
        pltpu.sync_copy(sref, dref)
    issue_chunk(1, cbufs[1])

    def pair(i, carry):
        carry = scan_chunk(2 * i, carry, cbufs[0][0], cbufs[0][1], cbufs[0][2])

        @pl.when(i < NSCH // 2 - 1)
        def _():
            issue_chunk(2 * i + 2, cbufs[0])
        wait_chunk(2 * i + 1, cbufs[1])
        carry = scan_chunk(2 * i + 1, carry,
                           cbufs[1][0], cbufs[1][1], cbufs[1][2])

        @pl.when(i < NSCH // 2 - 1)
        def _():
            issue_chunk(2 * i + 3, cbufs[1])
            wait_chunk(2 * i + 2, cbufs[0])
        return carry
    ptr, done = lax.fori_loop(0, NSCH // 2, pair, (0, 0))

    trash_l = jnp.full((16,), TRASH, jnp.int32)
    zero_i = jnp.zeros((16,), jnp.int32)
    zero_f = jnp.zeros((16,), jnp.float32)
    for t in range(DB // 16):
        ldsel[pl.ds(ptr + t * 16, 16)] = trash_l
        srcsel[pl.ds(ptr + t * 16, 16)] = zero_i
        wsel[pl.ds(ptr + t * 16, 16)] = zero_f
    _, done = flush((ptr, done))

    cbuf[...] = jnp.full((16,), done, jnp.int32)
    pltpu.sync_copy(cbuf, cnt_hbm.at[pl.ds(wid * 16, 16)])


@functools.partial(
    pl.kernel,
    out_type=(
        jax.ShapeDtypeStruct((NW * CAP,), jnp.int32),
        jax.ShapeDtypeStruct((NW * CAP,), jnp.int32),
        jax.ShapeDtypeStruct((NW * CAP,), jnp.float32),
        jax.ShapeDtypeStruct((NW * 16,), jnp.int32),
    ),
    mesh=plsc.VectorSubcoreMesh(**_SC_MESH),
    scratch_types=[
        pltpu.VMEM((SCH,), jnp.int32),
        pltpu.VMEM((SCH,), jnp.int32),
        pltpu.VMEM((SCH,), jnp.float32),
        pltpu.VMEM((SCH,), jnp.int32),
        pltpu.VMEM((SCH,), jnp.int32),
        pltpu.VMEM((SCH,), jnp.float32),
        pltpu.VMEM((DB + DB + 16,), jnp.int32),
        pltpu.VMEM((DB + DB + 16,), jnp.int32),
        pltpu.VMEM((DB + DB + 16,), jnp.float32),
        pltpu.VMEM((16,), jnp.int32),
        pltpu.SemaphoreType.DMA,
        pltpu.SemaphoreType.DMA,
    ],
)
def _edge_prep(*args):
    _prep_body(*args)


def _apply_body(h_hbm, ldst_hbm, srcl_hbm, wl_hbm, cnt_hbm, out_hbm,
                lbuf0, sbuf0, wbuf0, rows0, lbuf1, sbuf1, wbuf1, rows1,
                cbuf, acc, seml0, semr0, seml1, semr1):
    c = lax.axis_index("c")
    s = lax.axis_index("s")
    wid = c * NTL + s
    lo = wid * RPT
    lbase = wid * CAP

    pltpu.sync_copy(h_hbm.at[pl.ds(lo, RPT)], acc.at[pl.ds(0, RPT)])

    @pl.when(wid == NW - 1)
    def _():
        pltpu.sync_copy(h_hbm.at[pl.ds(lo + RPT, RLAST - RPT)],
                        acc.at[pl.ds(RPT, RLAST - RPT)])

    pltpu.sync_copy(cnt_hbm.at[pl.ds(wid * 16, 16)], cbuf)
    nb2 = cbuf[...][0] * (DB // DBA)

    bufs = ((lbuf0, sbuf0, wbuf0, rows0, seml0, semr0),
            (lbuf1, sbuf1, wbuf1, rows1, seml1, semr1))

    def lists_refs(b, bs):
        o = lbase + b * DBA
        return ((ldst_hbm.at[pl.ds(o, DBA)], bs[0]),
                (srcl_hbm.at[pl.ds(o, DBA)], bs[1]),
                (wl_hbm.at[pl.ds(o, DBA)], bs[2]))

    def issue_lists(b, bs):
        for sref, dref in lists_refs(b, bs):
            pltpu.async_copy(sref, dref, bs[4])

    def wait_lists(b, bs):
        for sref, dref in lists_refs(b, bs):
            pltpu.make_async_copy(sref, dref, bs[4]).wait()

    def process(b, this, other):
        @pl.when(b + 1 < nb2)
        def _():
            wait_lists(b + 1, other)
            pltpu.async_copy(h_hbm.at[other[1]], other[3], other[5])

        pltpu.make_async_copy(h_hbm.at[this[1]], this[3], this[5]).wait()
        rows, lb, wb = this[3], this[0], this[2]

        def acc16(gg, _):
            wg = wb[pl.ds(gg * 16, 16)]
            lg = lb[pl.ds(gg * 16, 16)]
            for j in range(16):
                r = lg[j]
                w = wg[j]
                e = gg * 16 + j
                vals = [acc[r, pl.ds(k * 16, 16)] + rows[e, pl.ds(k * 16, 16)] * w
                        for k in range(D // 16)]
                for k in range(D // 16):
                    acc[r, pl.ds(k * 16, 16)] = vals[k]
            return 0
        lax.fori_loop(0, DBA // 16, acc16, 0)

        @pl.when(b + 2 < nb2)
        def _():
            issue_lists(b + 2, this)

    for sref, dref in lists_refs(0, bufs[0]):
        pltpu.sync_copy(sref, dref)
    pltpu.async_copy(h_hbm.at[bufs[0][1]], bufs[0][3], bufs[0][5])

    @pl.when(nb2 > 1)
    def _():
        issue_lists(1, bufs[1])

    def pair(i, _):
        process(2 * i, bufs[0], bufs[1])

        @pl.when(2 * i + 1 < nb2)
        def _():
            process(2 * i + 1, bufs[1], bufs[0])
        return 0
    lax.fori_loop(0, (nb2 + 1) // 2, pair, 0)

    pltpu.sync_copy(acc.at[pl.ds(0, RPT)], out_hbm.at[pl.ds(lo, RPT)])

    @pl.when(wid == NW - 1)
    def _():
        pltpu.sync_copy(acc.at[pl.ds(RPT, RLAST - RPT)],
                        out_hbm.at[pl.ds(lo + RPT, RLAST - RPT)])


@functools.partial(
    pl.kernel,
    out_type=jax.ShapeDtypeStruct((N, D), jnp.float32),
    mesh=plsc.VectorSubcoreMesh(**_SC_MESH),
    scratch_types=[
        pltpu.VMEM((DBA,), jnp.int32),
        pltpu.VMEM((DBA,), jnp.int32),
        pltpu.VMEM((DBA,), jnp.float32),
        pltpu.VMEM((DBA, D), jnp.float32),
        pltpu.VMEM((DBA,), jnp.int32),
        pltpu.VMEM((DBA,), jnp.int32),
        pltpu.VMEM((DBA,), jnp.float32),
        pltpu.VMEM((DBA, D), jnp.float32),
        pltpu.VMEM((16,), jnp.int32),
        pltpu.VMEM((ACC_ROWS, D), jnp.float32),
        pltpu.SemaphoreType.DMA,
        pltpu.SemaphoreType.DMA,
        pltpu.SemaphoreType.DMA,
        pltpu.SemaphoreType.DMA,
    ],
)
def _spmm_apply(*args):
    _apply_body(*args)


def _mlp_stats_body(agg_ref, w0_ref, b0_ref, w1_ref, b1_ref, x_ref, stats_ref):
    a = agg_ref[...]
    t = jnp.maximum(jnp.dot(a, w0_ref[...], preferred_element_type=jnp.float32)
                    + b0_ref[...], 0.0)
    y = jnp.dot(t, w1_ref[...], preferred_element_type=jnp.float32) + b1_ref[...]
    y = jnp.where(y > 0, y, 0.01 * y)
    x_ref[...] = y

    @pl.when(pl.program_id(0) == 0)
    def _():
        stats_ref[...] = jnp.zeros_like(stats_ref)
    stats_ref[0:1, :] = stats_ref[0:1, :] + jnp.sum(y, axis=0, keepdims=True)
    stats_ref[1:2, :] = stats_ref[1:2, :] + jnp.sum(y * y, axis=0, keepdims=True)


def _mlp_stats(agg, w0, b0, w1, b1):
    return pl.pallas_call(
        _mlp_stats_body,
        grid=(NB,),
        in_specs=[
            pl.BlockSpec((RB, D), lambda i: (i, 0)),
            pl.BlockSpec((D, D), lambda i: (0, 0)),
            pl.BlockSpec((1, D), lambda i: (0, 0)),
            pl.BlockSpec((D, D), lambda i: (0, 0)),
            pl.BlockSpec((1, D), lambda i: (0, 0)),
        ],
        out_specs=[
            pl.BlockSpec((RB, D), lambda i: (i, 0)),
            pl.BlockSpec((8, D), lambda i: (0, 0)),
        ],
        out_shape=[
            jax.ShapeDtypeStruct((N, D), jnp.float32),
            jax.ShapeDtypeStruct((8, D), jnp.float32),
        ],
    )(agg, w0, b0, w1, b1)


def _bn_elin_body(x_ref, stats_ref, gamma_ref, beta_ref, attw_ref, sc_ref,
                  gp1_ref, gp2_ref, h_ref, e_ref, emax_ref):
    mean = stats_ref[0:1, :] * (1.0 / N)
    var = stats_ref[1:2, :] * (1.0 / N) - mean * mean
    inv = lax.rsqrt(var + 1e-5)
    h = gamma_ref[...] * (x_ref[...] - mean) * inv + beta_ref[...]
    h_ref[...] = h
    e = jnp.dot(h, attw_ref[...], preferred_element_type=jnp.float32)
    e = (e + gp1_ref[...] * sc_ref[0:1, 0:1] + gp2_ref[...] * sc_ref[0:1, 1:2]
         + sc_ref[0:1, 2:3])
    e_ref[...] = e

    @pl.when(pl.program_id(0) == 0)
    def _():
        emax_ref[...] = jnp.full_like(emax_ref, -jnp.inf)
    emax_ref[...] = jnp.maximum(emax_ref[...], jnp.max(e))


def _bn_elin(x, stats, gamma, beta, attw, sc, gp1, gp2):
    return pl.pallas_call(
        _bn_elin_body,
        grid=(NB,),
        in_specs=[
            pl.BlockSpec((RB, D), lambda i: (i, 0)),
            pl.BlockSpec((8, D), lambda i: (0, 0)),
            pl.BlockSpec((1, D), lambda i: (0, 0)),
            pl.BlockSpec((1, D), lambda i: (0, 0)),
            pl.BlockSpec((D, 1), lambda i: (0, 0)),
            pl.BlockSpec((1, 128), lambda i: (0, 0)),
            pl.BlockSpec((RB, 1), lambda i: (i, 0)),
            pl.BlockSpec((RB, 1), lambda i: (i, 0)),
        ],
        out_specs=[
            pl.BlockSpec((RB, D), lambda i: (i, 0)),
            pl.BlockSpec((RB, 1), lambda i: (i, 0)),
            pl.BlockSpec((1, 1), lambda i: (0, 0)),
        ],
        out_shape=[
            jax.ShapeDtypeStruct((N, D), jnp.float32),
            jax.ShapeDtypeStruct((N, 1), jnp.float32),
            jax.ShapeDtypeStruct((1, 1), jnp.float32),
        ],
    )(x, stats, gamma, beta, attw, sc, gp1, gp2)


def _elin_body(h_ref, attw_ref, sc_ref, gp1_ref, gp2_ref, e_ref, emax_ref):
    e = jnp.dot(h_ref[...], attw_ref[...], preferred_element_type=jnp.float32)
    e = (e + gp1_ref[...] * sc_ref[0:1, 0:1] + gp2_ref[...] * sc_ref[0:1, 1:2]
         + sc_ref[0:1, 2:3])
    e_ref[...] = e

    @pl.when(pl.program_id(0) == 0)
    def _():
        emax_ref[...] = jnp.full_like(emax_ref, -jnp.inf)
    emax_ref[...] = jnp.maximum(emax_ref[...], jnp.max(e))


def _elin(h, attw, sc, gp1, gp2):
    return pl.pallas_call(
        _elin_body,
        grid=(NB,),
        in_specs=[
            pl.BlockSpec((RB, D), lambda i: (i, 0)),
            pl.BlockSpec((D, 1), lambda i: (0, 0)),
            pl.BlockSpec((1, 128), lambda i: (0, 0)),
            pl.BlockSpec((RB, 1), lambda i: (i, 0)),
            pl.BlockSpec((RB, 1), lambda i: (i, 0)),
        ],
        out_specs=[
            pl.BlockSpec((RB, 1), lambda i: (i, 0)),
            pl.BlockSpec((1, 1), lambda i: (0, 0)),
        ],
        out_shape=[
            jax.ShapeDtypeStruct((N, 1), jnp.float32),
            jax.ShapeDtypeStruct((1, 1), jnp.float32),
        ],
    )(h, attw, sc, gp1, gp2)


def _pool_body(gid_ref, h0_ref, h1_ref, h2_ref, e0_ref, e1_ref, e2_ref,
               m0_ref, m1_ref, m2_ref,
               p0_ref, p1_ref, p2_ref, r0_ref, r1_ref, r2_ref):
    gid = gid_ref[0]
    oh = (gid == lax.broadcasted_iota(jnp.int32, (B, RB), 0)).astype(jnp.float32)

    @pl.when(pl.program_id(0) == 0)
    def _():
        for ref in (p0_ref, p1_ref, p2_ref, r0_ref, r1_ref, r2_ref):
            ref[...] = jnp.zeros_like(ref)

    for h_ref, e_ref, m_ref, p_ref, r_ref in (
            (h0_ref, e0_ref, m0_ref, p0_ref, r0_ref),
            (h1_ref, e1_ref, m1_ref, p1_ref, r1_ref),
            (h2_ref, e2_ref, m2_ref, p2_ref, r2_ref)):
        ee = jnp.exp(e_ref[...] - m_ref[...])
        eh = ee * h_ref[...]
        p_ref[...] = p_ref[...] + jnp.dot(oh, eh, preferred_element_type=jnp.float32)
        eb = jnp.broadcast_to(ee, (RB, 128))
        r_ref[...] = r_ref[...] + jnp.dot(oh, eb, preferred_element_type=jnp.float32)


def _pool(gid3, hs, es, ms):
    blk = lambda shape: pl.BlockSpec(shape, lambda i: (i, 0))
    cst = lambda shape: pl.BlockSpec(shape, lambda i: (0, 0))
    return pl.pallas_call(
        _pool_body,
        grid=(NB,),
        in_specs=[
            pl.BlockSpec((1, 1, RB), lambda i: (i, 0, 0)),
            blk((RB, D)), blk((RB, D)), blk((RB, D)),
            blk((RB, 1)), blk((RB, 1)), blk((RB, 1)),
            cst((1, 1)), cst((1, 1)), cst((1, 1)),
        ],
        out_specs=[cst((B, D)), cst((B, D)), cst((B, D)),
                   cst((B, 128)), cst((B, 128)), cst((B, 128))],
        out_shape=[jax.ShapeDtypeStruct((B, D), jnp.float32)] * 3
                  + [jax.ShapeDtypeStruct((B, 128), jnp.float32)] * 3,
    )(gid3, *hs, *es, *ms)


def _head_body(p0_ref, p1_ref, p2_ref, r0_ref, r1_ref, r2_ref,
               w0_ref, w1_ref, w2_ref, pb_ref,
               score_ref, o0_ref, o1_ref, o2_ref):
    score = jnp.zeros((B, OUT), jnp.float32)
    for i, (p_ref, r_ref, w_ref, o_ref) in enumerate(
            ((p0_ref, r0_ref, w0_ref, o0_ref),
             (p1_ref, r1_ref, w1_ref, o1_ref),
             (p2_ref, r2_ref, w2_ref, o2_ref))):
        pooled = p_ref[...] / (r_ref[:, 0:1] + 1e-10)
        o_ref[...] = pooled
        score = score + jnp.dot(pooled, w_ref[...],
                                preferred_element_type=jnp.float32) \
            + pb_ref[i:i + 1, :]
    score_ref[...] = score


def _head(praws, rsums, predws, predb):
    full = lambda shape: pl.BlockSpec(shape, lambda: (0, 0))
    return pl.pallas_call(
        _head_body,
        in_specs=[full((B, D))] * 3 + [full((B, 128))] * 3
                 + [full((D, OUT))] * 3 + [full((3, OUT))],
        out_specs=[full((B, OUT))] + [full((B, D))] * 3,
        out_shape=[jax.ShapeDtypeStruct((B, OUT), jnp.float32)]
                  + [jax.ShapeDtypeStruct((B, D), jnp.float32)] * 3,
    )(*praws, *rsums, *predws, predb)


def kernel(node_ids, pos_enc, edge_index, edge_weights, graph_ids, elem_gp1,
           elem_gp2, word_emb, pos, gnn_W0, gnn_b0, gnn_W1, gnn_b1, bn_gamma,
           bn_beta, att_W, att_b, pred_W, pred_b):
    src = edge_index[0]
    dst = edge_index[1]
    gp1 = elem_gp1.reshape(N, 1)
    gp2 = elem_gp2.reshape(N, 1)
    gid3 = graph_ids.reshape(NB, 1, RB)

    def att_params(l):
        attw = att_W[l, :D, :]
        sc = jnp.zeros((1, 128), jnp.float32)
        sc = sc.at[0, 0].set(att_W[l, D, 0])
        sc = sc.at[0, 1].set(att_W[l, D + 1, 0])
        sc = sc.at[0, 2].set(att_b[l, 0])
        return attw, sc

    pos16 = jnp.broadcast_to(pos[0:1], (16,))
    h = _embed(node_ids, pos_enc, word_emb, pos16)
    elist_ldst, elist_src, elist_w, elist_cnt = _edge_prep(src, dst,
                                                           edge_weights)

    attw0, sc0 = att_params(0)
    e0, m0 = _elin(h, attw0, sc0, gp1, gp2)

    hs, es, ms = [h], [e0], [m0]
    for l in range(2):
        agg = _spmm_apply(h, elist_ldst, elist_src, elist_w, elist_cnt)

        x, stats = _mlp_stats(agg, gnn_W0[l], gnn_b0[l].reshape(1, D),
                              gnn_W1[l], gnn_b1[l].reshape(1, D))
        attw, sc = att_params(l + 1)
        h, e, m = _bn_elin(x, stats, bn_gamma[l].reshape(1, D),
                           bn_beta[l].reshape(1, D), attw, sc, gp1, gp2)
        hs.append(h); es.append(e); ms.append(m)

    p0, p1, p2, r0, r1, r2 = _pool(gid3, hs, es, ms)
    score, o0, o1, o2 = _head((p0, p1, p2), (r0, r1, r2),
                              (pred_W[0], pred_W[1], pred_W[2]), pred_b)
    return (score, o0, o1, o2)


# speedup vs baseline: 2.3825x; 1.0220x over previous
import functools

import jax
import jax.numpy as jnp
from jax import lax
from jax.experimental import pallas as pl
from jax.experimental.pallas import tpu as pltpu
from jax.experimental.pallas import tpu_sc as plsc

N = 10000
E = 160000
D = 256
B = 16
OUT = 16
RB = 1000
NB = N // RB

NSC = 2
NTL = 16
NW = NSC * NTL
RPT = 312
RLAST = N - RPT * (NW - 1)
TRASH = 328
ACC_ROWS = 336
DB = 160
DBA = 80
CAP = E + DB
SCH = 1600
NSCH = E // SCH
NGRP = SCH // 64
ERC = 80
_SC_MESH = dict(core_axis_name="c", subcore_axis_name="s",
                num_cores=NSC, num_subcores=NTL)


def _embed_body(ids_hbm, pe_hbm, emb_hbm, p16_hbm, out_hbm,
                idxv, rows, pev, p16v, sem):
    c = lax.axis_index("c")
    s = lax.axis_index("s")
    wid = s * NSC + c
    start = jnp.minimum(wid * (4 * ERC), N - 4 * ERC)
    pltpu.sync_copy(p16_hbm, p16v)
    p0 = p16v[...]
    for j in range(4):
        o = start + j * ERC
        pltpu.sync_copy(ids_hbm.at[pl.ds(o, ERC)], idxv)
        pltpu.async_copy(emb_hbm.at[idxv], rows, sem).wait()
        pltpu.sync_copy(pe_hbm.at[pl.ds(o, ERC)], pev)

        def addrow(r, _):
            for k in range(D // 16):
                sl = pl.ds(k * 16, 16)
                rows[r, sl] = rows[r, sl] + p0 * pev[r, sl]
            return 0
        lax.fori_loop(0, ERC, addrow, 0)
        pltpu.sync_copy(rows, out_hbm.at[pl.ds(o, ERC)])


@functools.partial(
    pl.kernel,
    out_type=jax.ShapeDtypeStruct((N, D), jnp.float32),
    mesh=plsc.VectorSubcoreMesh(**_SC_MESH),
    scratch_types=[
        pltpu.VMEM((ERC,), jnp.int32),
        pltpu.VMEM((ERC, D), jnp.float32),
        pltpu.VMEM((ERC, D), jnp.float32),
        pltpu.VMEM((16,), jnp.float32),
        pltpu.SemaphoreType.DMA,
    ],
)
def _embed(*args):
    _embed_body(*args)


_GDN = lax.GatherDimensionNumbers(
    offset_dims=(), collapsed_slice_dims=(0,), start_index_map=(0,))


def _dg(vec, idx):
    return lax.gather(vec, idx[:, None], _GDN, (1,),
                      mode=lax.GatherScatterMode.PROMISE_IN_BOUNDS)


def _prep_body(src_hbm, dst_hbm, w_hbm,
               ldst_hbm, srcl_hbm, wl_hbm, cnt_hbm,
               dstb, srcb, wch, dstb1, srcb1, wch1, ldsel, srcsel, wsel, cbuf,
               semc0, semc1):
    c = lax.axis_index("c")
    s = lax.axis_index("s")
    wid = c * NTL + s
    lo = wid * RPT
    hi = lo + jnp.where(wid == NW - 1, RLAST, RPT)
    lbase = wid * CAP
    lane = lax.iota(jnp.int32, 16)

    def flush(carry):
        ptr, done = carry
        o = lbase + done * DB
        pltpu.sync_copy(ldsel.at[pl.ds(0, DB)], ldst_hbm.at[pl.ds(o, DB)])
        pltpu.sync_copy(srcsel.at[pl.ds(0, DB)], srcl_hbm.at[pl.ds(o, DB)])
        pltpu.sync_copy(wsel.at[pl.ds(0, DB)], wl_hbm.at[pl.ds(o, DB)])
        ldsel[pl.ds(0, 16)] = ldsel[pl.ds(DB, 16)]
        srcsel[pl.ds(0, 16)] = srcsel[pl.ds(DB, 16)]
        wsel[pl.ds(0, 16)] = wsel[pl.ds(DB, 16)]
        return ptr - DB, done + 1

    shidx = [jnp.maximum(lane - k, 0) for k in (1, 2, 4, 8)]
    shmask = [lane >= k for k in (1, 2, 4, 8)]
    zero16 = jnp.zeros((16,), jnp.int32)

    def scan_chunk(ch, carry, db, sb, wb):
        def prefix(m):
            x = jnp.where(m, 1, 0)
            for k in range(4):
                x = x + jnp.where(shmask[k], _dg(x, shidx[k]), 0)
            return x

        def mksel(v, x, cnt, sl16, sb, wb):
            def sel(carry):
                ptr, done = carry
                tgt = lane + 1
                pos = zero16
                for st in (8, 4, 2, 1):
                    cand = pos + st
                    xv = _dg(x, cand - 1)
                    pos = jnp.where(xv < tgt, cand, pos)
                srci = jnp.minimum(pos, 15)
                ldsel[pl.ds(ptr, 16)] = _dg(v, srci) - lo
                srcsel[pl.ds(ptr, 16)] = _dg(sb[sl16], srci)
                wsel[pl.ds(ptr, 16)] = _dg(wb[sl16], srci)
                ptr = ptr + cnt
                return lax.cond(ptr >= DB, flush, lambda cc: cc, (ptr, done))
            return sel

        def grp(g, carry):
            sls = [pl.ds(g * 64 + 16 * t, 16) for t in range(4)]
            vs = [db[sl] for sl in sls]
            ms = [(v >= lo) & (v < hi) for v in vs]
            xs = [prefix(m) for m in ms]
            cs = [x[15] for x in xs]
            for t in range(4):
                carry = lax.cond(cs[t] > 0,
                                 mksel(vs[t], xs[t], cs[t], sls[t], sb, wb),
                                 lambda cc: cc, carry)
            return carry
        return lax.fori_loop(0, NGRP, grp, carry)

    cbufs = ((dstb, srcb, wch, semc0), (dstb1, srcb1, wch1, semc1))

    def chunk_refs(ch, cb):
        csl = pl.ds(ch * SCH, SCH)
        return ((dst_hbm.at[csl], cb[0]), (src_hbm.at[csl], cb[1]),
                (w_hbm.at[csl], cb[2]))

    def issue_chunk(ch, cb):
        for sref, dref in chunk_refs(ch, cb):
            pltpu.async_copy(sref, dref, cb[3])

    def wait_chunk(ch, cb):
        for sref, dref in chunk_refs(ch, cb):
            pltpu.make_async_copy(sref, dref, cb[3]).wait()

    for sref, dref in chunk_refs(0, cbufs[0]):
        pltpu.sync_copy(sref, dref)
    issue_chunk(1, cbufs[1])

    def pair(i, carry):
        carry = scan_chunk(2 * i, carry, cbufs[0][0], cbufs[0][1], cbufs[0][2])

        @pl.when(i < NSCH // 2 - 1)
        def _():
            issue_chunk(2 * i + 2, cbufs[0])
        wait_chunk(2 * i + 1, cbufs[1])
        carry = scan_chunk(2 * i + 1, carry,
                           cbufs[1][0], cbufs[1][1], cbufs[1][2])

        @pl.when(i < NSCH // 2 - 1)
        def _():
            issue_chunk(2 * i + 3, cbufs[1])
            wait_chunk(2 * i + 2, cbufs[0])
        return carry
    ptr, done = lax.fori_loop(0, NSCH // 2, pair, (0, 0))

    trash_l = jnp.full((16,), TRASH, jnp.int32)
    zero_i = jnp.zeros((16,), jnp.int32)
    zero_f = jnp.zeros((16,), jnp.float32)
    for t in range(DB // 16):
        ldsel[pl.ds(ptr + t * 16, 16)] = trash_l
        srcsel[pl.ds(ptr + t * 16, 16)] = zero_i
        wsel[pl.ds(ptr + t * 16, 16)] = zero_f
    _, done = flush((ptr, done))

    cbuf[...] = jnp.full((16,), done, jnp.int32)
    pltpu.sync_copy(cbuf, cnt_hbm.at[pl.ds(wid * 16, 16)])


@functools.partial(
    pl.kernel,
    out_type=(
        jax.ShapeDtypeStruct((NW * CAP,), jnp.int32),
        jax.ShapeDtypeStruct((NW * CAP,), jnp.int32),
        jax.ShapeDtypeStruct((NW * CAP,), jnp.float32),
        jax.ShapeDtypeStruct((NW * 16,), jnp.int32),
    ),
    mesh=plsc.VectorSubcoreMesh(**_SC_MESH),
    scratch_types=[
        pltpu.VMEM((SCH,), jnp.int32),
        pltpu.VMEM((SCH,), jnp.int32),
        pltpu.VMEM((SCH,), jnp.float32),
        pltpu.VMEM((SCH,), jnp.int32),
        pltpu.VMEM((SCH,), jnp.int32),
        pltpu.VMEM((SCH,), jnp.float32),
        pltpu.VMEM((DB + DB + 16,), jnp.int32),
        pltpu.VMEM((DB + DB + 16,), jnp.int32),
        pltpu.VMEM((DB + DB + 16,), jnp.float32),
        pltpu.VMEM((16,), jnp.int32),
        pltpu.SemaphoreType.DMA,
        pltpu.SemaphoreType.DMA,
    ],
)
def _edge_prep(*args):
    _prep_body(*args)


def _apply_body(h_hbm, ldst_hbm, srcl_hbm, wl_hbm, cnt_hbm, out_hbm,
                lbuf0, sbuf0, wbuf0, rows0, lbuf1, sbuf1, wbuf1, rows1,
                cbuf, acc, seml0, semr0, seml1, semr1):
    c = lax.axis_index("c")
    s = lax.axis_index("s")
    wid = c * NTL + s
    lo = wid * RPT
    lbase = wid * CAP

    pltpu.sync_copy(h_hbm.at[pl.ds(lo, RPT)], acc.at[pl.ds(0, RPT)])

    @pl.when(wid == NW - 1)
    def _():
        pltpu.sync_copy(h_hbm.at[pl.ds(lo + RPT, RLAST - RPT)],
                        acc.at[pl.ds(RPT, RLAST - RPT)])

    pltpu.sync_copy(cnt_hbm.at[pl.ds(wid * 16, 16)], cbuf)
    nb2 = cbuf[...][0] * (DB // DBA)

    bufs = ((lbuf0, sbuf0, wbuf0, rows0, seml0, semr0),
            (lbuf1, sbuf1, wbuf1, rows1, seml1, semr1))

    def lists_refs(b, bs):
        o = lbase + b * DBA
        return ((ldst_hbm.at[pl.ds(o, DBA)], bs[0]),
                (srcl_hbm.at[pl.ds(o, DBA)], bs[1]),
                (wl_hbm.at[pl.ds(o, DBA)], bs[2]))

    def issue_lists(b, bs):
        for sref, dref in lists_refs(b, bs):
            pltpu.async_copy(sref, dref, bs[4])

    def wait_lists(b, bs):
        for sref, dref in lists_refs(b, bs):
            pltpu.make_async_copy(sref, dref, bs[4]).wait()

    def process(b, this, other):
        @pl.when(b + 1 < nb2)
        def _():
            wait_lists(b + 1, other)
            pltpu.async_copy(h_hbm.at[other[1]], other[3], other[5])

        pltpu.make_async_copy(h_hbm.at[this[1]], this[3], this[5]).wait()
        rows, lb, wb = this[3], this[0], this[2]

        def acc16(gg, _):
            wg = wb[pl.ds(gg * 16, 16)]
            lg = lb[pl.ds(gg * 16, 16)]
            for j in range(16):
                r = lg[j]
                w = wg[j]
                e = gg * 16 + j
                vals = [acc[r, pl.ds(k * 16, 16)] + rows[e, pl.ds(k * 16, 16)] * w
                        for k in range(D // 16)]
                for k in range(D // 16):
                    acc[r, pl.ds(k * 16, 16)] = vals[k]
            return 0
        lax.fori_loop(0, DBA // 16, acc16, 0)

        @pl.when(b + 2 < nb2)
        def _():
            issue_lists(b + 2, this)

    for sref, dref in lists_refs(0, bufs[0]):
        pltpu.sync_copy(sref, dref)
    pltpu.async_copy(h_hbm.at[bufs[0][1]], bufs[0][3], bufs[0][5])

    @pl.when(nb2 > 1)
    def _():
        issue_lists(1, bufs[1])

    def pair(i, _):
        process(2 * i, bufs[0], bufs[1])

        @pl.when(2 * i + 1 < nb2)
        def _():
            process(2 * i + 1, bufs[1], bufs[0])
        return 0
    lax.fori_loop(0, (nb2 + 1) // 2, pair, 0)

    pltpu.sync_copy(acc.at[pl.ds(0, RPT)], out_hbm.at[pl.ds(lo, RPT)])

    @pl.when(wid == NW - 1)
    def _():
        pltpu.sync_copy(acc.at[pl.ds(RPT, RLAST - RPT)],
                        out_hbm.at[pl.ds(lo + RPT, RLAST - RPT)])


@functools.partial(
    pl.kernel,
    out_type=jax.ShapeDtypeStruct((N, D), jnp.float32),
    mesh=plsc.VectorSubcoreMesh(**_SC_MESH),
    scratch_types=[
        pltpu.VMEM((DBA,), jnp.int32),
        pltpu.VMEM((DBA,), jnp.int32),
        pltpu.VMEM((DBA,), jnp.float32),
        pltpu.VMEM((DBA, D), jnp.float32),
        pltpu.VMEM((DBA,), jnp.int32),
        pltpu.VMEM((DBA,), jnp.int32),
        pltpu.VMEM((DBA,), jnp.float32),
        pltpu.VMEM((DBA, D), jnp.float32),
        pltpu.VMEM((16,), jnp.int32),
        pltpu.VMEM((ACC_ROWS, D), jnp.float32),
        pltpu.SemaphoreType.DMA,
        pltpu.SemaphoreType.DMA,
        pltpu.SemaphoreType.DMA,
        pltpu.SemaphoreType.DMA,
    ],
)
def _spmm_apply(*args):
    _apply_body(*args)


def _mlp_stats_body(agg_ref, w0_ref, b0_ref, w1_ref, b1_ref, x_ref, stats_ref):
    a = agg_ref[...]
    t = jnp.maximum(jnp.dot(a, w0_ref[...], preferred_element_type=jnp.float32)
                    + b0_ref[...], 0.0)
    y = jnp.dot(t, w1_ref[...], preferred_element_type=jnp.float32) + b1_ref[...]
    y = jnp.where(y > 0, y, 0.01 * y)
    x_ref[...] = y

    @pl.when(pl.program_id(0) == 0)
    def _():
        stats_ref[...] = jnp.zeros_like(stats_ref)
    stats_ref[0:1, :] = stats_ref[0:1, :] + jnp.sum(y, axis=0, keepdims=True)
    stats_ref[1:2, :] = stats_ref[1:2, :] + jnp.sum(y * y, axis=0, keepdims=True)


def _mlp_stats(agg, w0, b0, w1, b1):
    return pl.pallas_call(
        _mlp_stats_body,
        grid=(NB,),
        in_specs=[
            pl.BlockSpec((RB, D), lambda i: (i, 0)),
            pl.BlockSpec((D, D), lambda i: (0, 0)),
            pl.BlockSpec((1, D), lambda i: (0, 0)),
            pl.BlockSpec((D, D), lambda i: (0, 0)),
            pl.BlockSpec((1, D), lambda i: (0, 0)),
        ],
        out_specs=[
            pl.BlockSpec((RB, D), lambda i: (i, 0)),
            pl.BlockSpec((8, D), lambda i: (0, 0)),
        ],
        out_shape=[
            jax.ShapeDtypeStruct((N, D), jnp.float32),
            jax.ShapeDtypeStruct((8, D), jnp.float32),
        ],
    )(agg, w0, b0, w1, b1)


def _bn_elin_body(x_ref, stats_ref, gamma_ref, beta_ref, attw_ref, sc_ref,
                  gp1_ref, gp2_ref, h_ref, e_ref, emax_ref):
    mean = stats_ref[0:1, :] * (1.0 / N)
    var = stats_ref[1:2, :] * (1.0 / N) - mean * mean
    inv = lax.rsqrt(var + 1e-5)
    h = gamma_ref[...] * (x_ref[...] - mean) * inv + beta_ref[...]
    h_ref[...] = h
    e = jnp.dot(h, attw_ref[...], preferred_element_type=jnp.float32)
    e = (e + gp1_ref[...] * sc_ref[0:1, 0:1] + gp2_ref[...] * sc_ref[0:1, 1:2]
         + sc_ref[0:1, 2:3])
    e_ref[...] = e

    @pl.when(pl.program_id(0) == 0)
    def _():
        emax_ref[...] = jnp.full_like(emax_ref, -jnp.inf)
    emax_ref[...] = jnp.maximum(emax_ref[...], jnp.max(e))


def _bn_elin(x, stats, gamma, beta, attw, sc, gp1, gp2):
    return pl.pallas_call(
        _bn_elin_body,
        grid=(NB,),
        in_specs=[
            pl.BlockSpec((RB, D), lambda i: (i, 0)),
            pl.BlockSpec((8, D), lambda i: (0, 0)),
            pl.BlockSpec((1, D), lambda i: (0, 0)),
            pl.BlockSpec((1, D), lambda i: (0, 0)),
            pl.BlockSpec((D, 1), lambda i: (0, 0)),
            pl.BlockSpec((1, 128), lambda i: (0, 0)),
            pl.BlockSpec((RB, 1), lambda i: (i, 0)),
            pl.BlockSpec((RB, 1), lambda i: (i, 0)),
        ],
        out_specs=[
            pl.BlockSpec((RB, D), lambda i: (i, 0)),
            pl.BlockSpec((RB, 1), lambda i: (i, 0)),
            pl.BlockSpec((1, 1), lambda i: (0, 0)),
        ],
        out_shape=[
            jax.ShapeDtypeStruct((N, D), jnp.float32),
            jax.ShapeDtypeStruct((N, 1), jnp.float32),
            jax.ShapeDtypeStruct((1, 1), jnp.float32),
        ],
    )(x, stats, gamma, beta, attw, sc, gp1, gp2)


def _elin_body(h_ref, attw_ref, sc_ref, gp1_ref, gp2_ref, e_ref, emax_ref):
    e = jnp.dot(h_ref[...], attw_ref[...], preferred_element_type=jnp.float32)
    e = (e + gp1_ref[...] * sc_ref[0:1, 0:1] + gp2_ref[...] * sc_ref[0:1, 1:2]
         + sc_ref[0:1, 2:3])
    e_ref[...] = e

    @pl.when(pl.program_id(0) == 0)
    def _():
        emax_ref[...] = jnp.full_like(emax_ref, -jnp.inf)
    emax_ref[...] = jnp.maximum(emax_ref[...], jnp.max(e))


def _elin(h, attw, sc, gp1, gp2):
    return pl.pallas_call(
        _elin_body,
        grid=(NB,),
        in_specs=[
            pl.BlockSpec((RB, D), lambda i: (i, 0)),
            pl.BlockSpec((D, 1), lambda i: (0, 0)),
            pl.BlockSpec((1, 128), lambda i: (0, 0)),
            pl.BlockSpec((RB, 1), lambda i: (i, 0)),
            pl.BlockSpec((RB, 1), lambda i: (i, 0)),
        ],
        out_specs=[
            pl.BlockSpec((RB, 1), lambda i: (i, 0)),
            pl.BlockSpec((1, 1), lambda i: (0, 0)),
        ],
        out_shape=[
            jax.ShapeDtypeStruct((N, 1), jnp.float32),
            jax.ShapeDtypeStruct((1, 1), jnp.float32),
        ],
    )(h, attw, sc, gp1, gp2)


def _pool_body(gid_ref, h0_ref, h1_ref, h2_ref, e0_ref, e1_ref, e2_ref,
               m0_ref, m1_ref, m2_ref,
               p0_ref, p1_ref, p2_ref, r0_ref, r1_ref, r2_ref):
    gid = gid_ref[0]
    oh = (gid == lax.broadcasted_iota(jnp.int32, (B, RB), 0)).astype(jnp.float32)

    @pl.when(pl.program_id(0) == 0)
    def _():
        for ref in (p0_ref, p1_ref, p2_ref, r0_ref, r1_ref, r2_ref):
            ref[...] = jnp.zeros_like(ref)

    for h_ref, e_ref, m_ref, p_ref, r_ref in (
            (h0_ref, e0_ref, m0_ref, p0_ref, r0_ref),
            (h1_ref, e1_ref, m1_ref, p1_ref, r1_ref),
            (h2_ref, e2_ref, m2_ref, p2_ref, r2_ref)):
        ee = jnp.exp(e_ref[...] - m_ref[...])
        eh = ee * h_ref[...]
        p_ref[...] = p_ref[...] + jnp.dot(oh, eh, preferred_element_type=jnp.float32)
        eb = jnp.broadcast_to(ee, (RB, 128))
        r_ref[...] = r_ref[...] + jnp.dot(oh, eb, preferred_element_type=jnp.float32)


def _pool(gid3, hs, es, ms):
    blk = lambda shape: pl.BlockSpec(shape, lambda i: (i, 0))
    cst = lambda shape: pl.BlockSpec(shape, lambda i: (0, 0))
    return pl.pallas_call(
        _pool_body,
        grid=(NB,),
        in_specs=[
            pl.BlockSpec((1, 1, RB), lambda i: (i, 0, 0)),
            blk((RB, D)), blk((RB, D)), blk((RB, D)),
            blk((RB, 1)), blk((RB, 1)), blk((RB, 1)),
            cst((1, 1)), cst((1, 1)), cst((1, 1)),
        ],
        out_specs=[cst((B, D)), cst((B, D)), cst((B, D)),
                   cst((B, 128)), cst((B, 128)), cst((B, 128))],
        out_shape=[jax.ShapeDtypeStruct((B, D), jnp.float32)] * 3
                  + [jax.ShapeDtypeStruct((B, 128), jnp.float32)] * 3,
    )(gid3, *hs, *es, *ms)


def _head_body(p0_ref, p1_ref, p2_ref, r0_ref, r1_ref, r2_ref,
               w0_ref, w1_ref, w2_ref, pb_ref,
               score_ref, o0_ref, o1_ref, o2_ref):
    score = jnp.zeros((B, OUT), jnp.float32)
    for i, (p_ref, r_ref, w_ref, o_ref) in enumerate(
            ((p0_ref, r0_ref, w0_ref, o0_ref),
             (p1_ref, r1_ref, w1_ref, o1_ref),
             (p2_ref, r2_ref, w2_ref, o2_ref))):
        pooled = p_ref[...] / (r_ref[:, 0:1] + 1e-10)
        o_ref[...] = pooled
        score = score + jnp.dot(pooled, w_ref[...],
                                preferred_element_type=jnp.float32) \
            + pb_ref[i:i + 1, :]
    score_ref[...] = score


def _head(praws, rsums, predws, predb):
    full = lambda shape: pl.BlockSpec(shape, lambda: (0, 0))
    return pl.pallas_call(
        _head_body,
        in_specs=[full((B, D))] * 3 + [full((B, 128))] * 3
                 + [full((D, OUT))] * 3 + [full((3, OUT))],
        out_specs=[full((B, OUT))] + [full((B, D))] * 3,
        out_shape=[jax.ShapeDtypeStruct((B, OUT), jnp.float32)]
                  + [jax.ShapeDtypeStruct((B, D), jnp.float32)] * 3,
    )(*praws, *rsums, *predws, predb)


def kernel(node_ids, pos_enc, edge_index, edge_weights, graph_ids, elem_gp1,
           elem_gp2, word_emb, pos, gnn_W0, gnn_b0, gnn_W1, gnn_b1, bn_gamma,
           bn_beta, att_W, att_b, pred_W, pred_b):
    src = edge_index[0]
    dst = edge_index[1]
    gp1 = elem_gp1.reshape(N, 1)
    gp2 = elem_gp2.reshape(N, 1)
    gid3 = graph_ids.reshape(NB, 1, RB)

    def att_params(l):
        attw = att_W[l, :D, :]
        sc = jnp.zeros((1, 128), jnp.float32)
        sc = sc.at[0, 0].set(att_W[l, D, 0])
        sc = sc.at[0, 1].set(att_W[l, D + 1, 0])
        sc = sc.at[0, 2].set(att_b[l, 0])
        return attw, sc

    pos16 = jnp.broadcast_to(pos[0:1], (16,))
    h = _embed(node_ids, pos_enc, word_emb, pos16)
    elist_ldst, elist_src, elist_w, elist_cnt = _edge_prep(src, dst,
                                                           edge_weights)

    attw0, sc0 = att_params(0)
    e0, m0 = _elin(h, attw0, sc0, gp1, gp2)

    hs, es, ms = [h], [e0], [m0]
    for l in range(2):
        agg = _spmm_apply(h, elist_ldst, elist_src, elist_w, elist_cnt)

        x, stats = _mlp_stats(agg, gnn_W0[l], gnn_b0[l].reshape(1, D),
                              gnn_W1[l], gnn_b1[l].reshape(1, D))
        attw, sc = att_params(l + 1)
        h, e, m = _bn_elin(x, stats, bn_gamma[l].reshape(1, D),
                           bn_beta[l].reshape(1, D), attw, sc, gp1, gp2)
        hs.append(h); es.append(e); ms.append(m)

    p0, p1, p2, r0, r1, r2 = _pool(gid3, hs, es, ms)
    score, o0, o1, o2 = _head((p0, p1, p2), (r0, r1, r2),
                              (pred_W[0], pred_W[1], pred_W[2]), pred_b)
    return (score, o0, o1, o2)
